# Initial kernel scaffold; baseline (speedup 1.0000x reference)
import jax
import jax.numpy as jnp
from jax.experimental import pallas as pl


def kernel(pts):
    raise NotImplementedError("write your pallas kernel here")


# speedup vs baseline: 1.4926x; 1.4926x over previous
import functools

import jax
import jax.numpy as jnp
import numpy as np
from jax import lax
from jax.experimental import pallas as pl
from jax.experimental.pallas import tpu as pltpu
from jax.experimental.pallas import tpu_sc as plsc

_B = 64
_N = 16384
_M = 13107
_MP = 13184


def _rotation_matrices(key, b):
    ang = jax.random.uniform(key, (b, 3), minval=0.0, maxval=2.0 * np.pi)
    ca, sa = jnp.cos(ang), jnp.sin(ang)
    cx, cy, cz = ca[:, 0], ca[:, 1], ca[:, 2]
    sx, sy, sz = sa[:, 0], sa[:, 1], sa[:, 2]
    one = jnp.ones_like(cx)
    zero = jnp.zeros_like(cx)
    Rx = jnp.stack([jnp.stack([one, zero, zero], -1),
                    jnp.stack([zero, cx, -sx], -1),
                    jnp.stack([zero, sx, cx], -1)], -2)
    Ry = jnp.stack([jnp.stack([cy, zero, sy], -1),
                    jnp.stack([zero, one, zero], -1),
                    jnp.stack([-sy, zero, cy], -1)], -2)
    Rz = jnp.stack([jnp.stack([cz, -sz, zero], -1),
                    jnp.stack([sz, cz, zero], -1),
                    jnp.stack([zero, zero, one], -1)], -2)
    return jnp.einsum('bij,bjk,bkl->bil', Rz, Ry, Rx)


def _dist_body(centers_ref, pts_ref, d_ref):
    b = pl.program_id(0)
    for crop in range(2):
        cx = centers_ref[b, crop, 0]
        cy = centers_ref[b, crop, 1]
        cz = centers_ref[b, crop, 2]
        dx = pts_ref[0, 0:1, :] - cx
        dy = pts_ref[0, 1:2, :] - cy
        dz = pts_ref[0, 2:3, :] - cz
        d = jnp.sqrt(dx * dx + dy * dy + dz * dz)
        d_ref[0, crop:crop + 1, :] = d


def _distances(pts_t, centers):
    return pl.pallas_call(
        _dist_body,
        grid=(_B,),
        in_specs=[
            pl.BlockSpec(memory_space=pltpu.SMEM),
            pl.BlockSpec((1, 3, _N), lambda b: (b, 0, 0)),
        ],
        out_specs=pl.BlockSpec((1, 2, _N), lambda b: (b, 0, 0)),
        out_shape=jax.ShapeDtypeStruct((_B, 2, _N), jnp.float32),
    )(centers, pts_t)


def _finalize_body(rot_ref, sel_ref, view_ref, viewr_ref, rel_ref):
    b = pl.program_id(0)
    lane = lax.broadcasted_iota(jnp.int32, (1, _MP), 1)
    valid = lane < _M
    means = []
    for crop in range(2):
        rows = []
        msum = []
        for comp in range(3):
            row = sel_ref[0, crop, comp:comp + 1, :]
            s = jnp.sum(jnp.where(valid, row, 0.0))
            m = s * (1.0 / _M)
            msum.append(m)
            rows.append(row - m)
        means.append(msum)
        n2 = rows[0] * rows[0] + rows[1] * rows[1] + rows[2] * rows[2]
        r2 = jnp.max(jnp.where(valid, n2, 0.0))
        denom = jnp.sqrt(r2) + 1e-12
        vrows = [r / denom for r in rows]
        for comp in range(3):
            view_ref[0, crop, comp:comp + 1, :] = vrows[comp]
        for e in range(3):
            vr = (vrows[0] * rot_ref[b, crop, 0, e]
                  + vrows[1] * rot_ref[b, crop, 1, e]
                  + vrows[2] * rot_ref[b, crop, 2, e])
            viewr_ref[0, crop, e:e + 1, :] = vr
    col = lax.broadcasted_iota(jnp.int32, (1, 8, 128), 2)
    rel = jnp.where(
        col == 0, means[1][0] - means[0][0],
        jnp.where(col == 1, means[1][1] - means[0][1],
                  means[1][2] - means[0][2]))
    rel_ref[0, :, :] = rel[0]


def _finalize(sel, rots):
    return pl.pallas_call(
        _finalize_body,
        grid=(_B,),
        in_specs=[
            pl.BlockSpec(memory_space=pltpu.SMEM),
            pl.BlockSpec((1, 2, 3, _MP), lambda b: (b, 0, 0, 0)),
        ],
        out_specs=[
            pl.BlockSpec((1, 2, 3, _MP), lambda b: (b, 0, 0, 0)),
            pl.BlockSpec((1, 2, 3, _MP), lambda b: (b, 0, 0, 0)),
            pl.BlockSpec((1, 8, 128), lambda b: (b, 0, 0)),
        ],
        out_shape=[
            jax.ShapeDtypeStruct((_B, 2, 3, _MP), jnp.float32),
            jax.ShapeDtypeStruct((_B, 2, 3, _MP), jnp.float32),
            jax.ShapeDtypeStruct((_B, 8, 128), jnp.float32),
        ],
    )(rots, sel)


def _sort_gather(d, pts_t):
    _, idx = lax.top_k(-d, _MP)
    sel = jnp.take_along_axis(
        pts_t[:, None, :, :], idx[:, :, None, :], axis=3)
    return sel


def kernel(pts):
    base = jax.random.key(42)
    k1, k2, k3, k4 = jax.random.split(base, 4)
    ci1 = jax.random.randint(k1, (_B,), 0, _N)
    ci2 = jax.random.randint(k2, (_B,), 0, _N)
    rot1 = _rotation_matrices(k3, _B)
    rot2 = _rotation_matrices(k4, _B)
    rots = jnp.stack([rot1, rot2], axis=1)

    c1 = jnp.take_along_axis(
        pts, jnp.broadcast_to(ci1[:, None, None], (_B, 1, 3)), axis=1)
    c2 = jnp.take_along_axis(
        pts, jnp.broadcast_to(ci2[:, None, None], (_B, 1, 3)), axis=1)
    centers = jnp.concatenate([c1, c2], axis=1)

    pts_t = pts.transpose(0, 2, 1)
    d = _distances(pts_t, centers)
    sel = _sort_gather(d, pts_t)
    view, viewr, rel = _finalize(sel, rots)

    relative_center = rel[:, 0, :3]
    view1 = view[:, 0, :, :_M].transpose(0, 2, 1)
    view2 = view[:, 1, :, :_M].transpose(0, 2, 1)
    view1_rot = viewr[:, 0, :, :_M].transpose(0, 2, 1)
    view2_rot = viewr[:, 1, :, :_M].transpose(0, 2, 1)
    return (relative_center, (view1_rot, view1), (view2_rot, view2))


# speedup vs baseline: 1.7377x; 1.1642x over previous
import functools

import jax
import jax.numpy as jnp
import numpy as np
from jax import lax
from jax.experimental import pallas as pl
from jax.experimental.pallas import tpu as pltpu
from jax.experimental.pallas import tpu_sc as plsc

_B = 64
_N = 16384
_M = 13107
_PM = 13120
_PMV = _PM // 16
_CVV = 82
_NCH = _PMV // _CVV
_CW = _CVV * 48
_RW = _PM * 3
_NV = _N // 16


def _rotation_matrices(key, b):
    ang = jax.random.uniform(key, (b, 3), minval=0.0, maxval=2.0 * np.pi)
    ca, sa = jnp.cos(ang), jnp.sin(ang)
    cx, cy, cz = ca[:, 0], ca[:, 1], ca[:, 2]
    sx, sy, sz = sa[:, 0], sa[:, 1], sa[:, 2]
    one = jnp.ones_like(cx)
    zero = jnp.zeros_like(cx)
    Rx = jnp.stack([jnp.stack([one, zero, zero], -1),
                    jnp.stack([zero, cx, -sx], -1),
                    jnp.stack([zero, sx, cx], -1)], -2)
    Ry = jnp.stack([jnp.stack([cy, zero, sy], -1),
                    jnp.stack([zero, one, zero], -1),
                    jnp.stack([-sy, zero, cy], -1)], -2)
    Rz = jnp.stack([jnp.stack([cz, -sz, zero], -1),
                    jnp.stack([sz, cz, zero], -1),
                    jnp.stack([zero, zero, one], -1)], -2)
    return jnp.einsum('bij,bjk,bkl->bil', Rz, Ry, Rx)


def _dist_body(centers_ref, pts_ref, d_ref):
    b = pl.program_id(0)
    for crop in range(2):
        cx = centers_ref[b, crop, 0]
        cy = centers_ref[b, crop, 1]
        cz = centers_ref[b, crop, 2]
        dx = pts_ref[0, 0:1, :] - cx
        dy = pts_ref[0, 1:2, :] - cy
        dz = pts_ref[0, 2:3, :] - cz
        d = jnp.sqrt(dx * dx + dy * dy + dz * dz)
        d_ref[0, crop:crop + 1, :] = d


def _distances(pts_t, centers):
    return pl.pallas_call(
        _dist_body,
        grid=(_B,),
        in_specs=[
            pl.BlockSpec(memory_space=pltpu.SMEM),
            pl.BlockSpec((1, 3, _N), lambda b: (b, 0, 0)),
        ],
        out_specs=pl.BlockSpec((1, 2, _N), lambda b: (b, 0, 0)),
        out_shape=jax.ShapeDtypeStruct((_B, 2, _N), jnp.float32),
    )(centers, pts_t)


def _sc_body(dkeys_hbm, pts_hbm, crow_hbm, view_hbm, viewr_hbm, means_hbm,
             key_a, key_b, val_a, val_b, ptsx, ptsy, ptsz, hist,
             stage_v, stage_r, cbuf, rotbuf):
    lane = lax.broadcasted_iota(jnp.int32, (16,), 0)
    ones = jnp.ones((16,), jnp.int32)
    fzero = jnp.zeros((16,), jnp.float32)
    wid = lax.axis_index("s") * 2 + lax.axis_index("c")

    def radix_pass(kin, vin, kout, vout, shift, last):
        @pl.loop(0, 256, unroll=4)
        def _(i):
            hist[pl.ds(i * 16, 16)] = jnp.zeros((16,), jnp.int32)

        @pl.loop(0, _NV, unroll=4)
        def _(v):
            k = kin[pl.ds(v * 16, 16)]
            dig = lax.shift_right_logical(k, shift) & 255
            plsc.addupdate_scatter(hist, [dig * 16 + lane], ones)

        @pl.loop(0, 256, init_carry=jnp.int32(0), unroll=2)
        def _(i, carry):
            h = hist[pl.ds(i * 16, 16)]
            c = plsc.cumsum(h)
            hist[pl.ds(i * 16, 16)] = c - h + carry
            return carry + jnp.sum(h)

        @pl.loop(0, _NV, unroll=4)
        def _(v):
            k = kin[pl.ds(v * 16, 16)]
            val = vin[pl.ds(v * 16, 16)]
            dig = lax.shift_right_logical(k, shift) & 255
            addr = dig * 16 + lane
            pos = plsc.load_gather(hist, [addr])
            plsc.addupdate_scatter(hist, [addr], ones)
            if last:
                plsc.store_scatter(vout, [pos], val)
            else:
                paddr = (pos & 1023) * 16 + lax.shift_right_logical(pos, 10)
                plsc.store_scatter(kout, [paddr], k)
                plsc.store_scatter(vout, [paddr], val)

    for j in range(2):
        b = wid * 2 + j
        pltpu.sync_copy(pts_hbm.at[pl.ds((b * 3 + 0) * _N, _N)], ptsx)
        pltpu.sync_copy(pts_hbm.at[pl.ds((b * 3 + 1) * _N, _N)], ptsy)
        pltpu.sync_copy(pts_hbm.at[pl.ds((b * 3 + 2) * _N, _N)], ptsz)
        for crop in range(2):
            r = b * 2 + crop
            pltpu.sync_copy(dkeys_hbm.at[pl.ds(r * _N, _N)], key_b)
            pltpu.sync_copy(crow_hbm.at[pl.ds(r * 144, 144)], rotbuf)

            @pl.loop(0, _NV, unroll=4)
            def _(v):
                src = lane * 1024 + ((v + lane) & 1023)
                vec = plsc.load_gather(key_b, [src])
                plsc.store_scatter(key_a, [((v + lane) & 1023) * 16 + lane],
                                   vec)

            @pl.loop(0, _NV, unroll=4)
            def _(v):
                val_a[pl.ds(v * 16, 16)] = lane * 1024 + v

            radix_pass(key_a, val_a, key_b, val_b, 0, False)
            radix_pass(key_b, val_b, key_a, val_a, 8, False)
            radix_pass(key_a, val_a, key_b, val_b, 16, False)
            radix_pass(key_b, val_b, None, val_a, 24, True)

            @pl.loop(0, _PMV, init_carry=(fzero, fzero, fzero), unroll=4)
            def s1(v, carry):
                sx, sy, sz = carry
                idx = val_a[pl.ds(v * 16, 16)]
                x = plsc.load_gather(ptsx, [idx])
                y = plsc.load_gather(ptsy, [idx])
                z = plsc.load_gather(ptsz, [idx])
                key_a[pl.ds(v * 16, 16)] = plsc.bitcast(x, jnp.int32)
                key_b[pl.ds(v * 16, 16)] = plsc.bitcast(y, jnp.int32)
                val_b[pl.ds(v * 16, 16)] = plsc.bitcast(z, jnp.int32)
                valid = (v * 16 + lane) < _M
                return (sx + jnp.where(valid, x, 0.0),
                        sy + jnp.where(valid, y, 0.0),
                        sz + jnp.where(valid, z, 0.0))

            sx, sy, sz = s1
            mean_x = jnp.sum(sx) * (1.0 / _M)
            mean_y = jnp.sum(sy) * (1.0 / _M)
            mean_z = jnp.sum(sz) * (1.0 / _M)

            cbuf[pl.ds(0, 16)] = jnp.where(
                lane == 0, mean_x, jnp.where(lane == 1, mean_y, mean_z))
            pltpu.sync_copy(cbuf, means_hbm.at[pl.ds(r * 16, 16)])

            @pl.loop(0, _PMV, init_carry=fzero, unroll=4)
            def s2(v, carry):
                x = plsc.bitcast(key_a[pl.ds(v * 16, 16)], jnp.float32)
                x = x - mean_x
                y = plsc.bitcast(key_b[pl.ds(v * 16, 16)], jnp.float32)
                y = y - mean_y
                z = plsc.bitcast(val_b[pl.ds(v * 16, 16)], jnp.float32)
                z = z - mean_z
                n2 = x * x + y * y + z * z
                valid = (v * 16 + lane) < _M
                return jnp.maximum(carry, jnp.where(valid, n2, 0.0))

            r2 = jnp.max(s2)
            r2v = fzero + r2
            yv = plsc.bitcast(
                jnp.int32(0x5F3759DF)
                - lax.shift_right_logical(plsc.bitcast(r2v, jnp.int32), 1),
                jnp.float32)
            yv = yv * (1.5 - 0.5 * r2v * yv * yv)
            yv = yv * (1.5 - 0.5 * r2v * yv * yv)
            yv = yv * (1.5 - 0.5 * r2v * yv * yv)
            inv = 1.0 / (r2v * yv + 1e-12)

            r00 = rotbuf[pl.ds(0, 16)]
            r01 = rotbuf[pl.ds(16, 16)]
            r02 = rotbuf[pl.ds(32, 16)]
            r10 = rotbuf[pl.ds(48, 16)]
            r11 = rotbuf[pl.ds(64, 16)]
            r12 = rotbuf[pl.ds(80, 16)]
            r20 = rotbuf[pl.ds(96, 16)]
            r21 = rotbuf[pl.ds(112, 16)]
            r22 = rotbuf[pl.ds(128, 16)]

            for ch in range(_NCH):
                @pl.loop(0, _CVV, unroll=2)
                def _(v):
                    gv = ch * _CVV + v
                    x = plsc.bitcast(key_a[pl.ds(gv * 16, 16)], jnp.float32)
                    x = (x - mean_x) * inv
                    y = plsc.bitcast(key_b[pl.ds(gv * 16, 16)], jnp.float32)
                    y = (y - mean_y) * inv
                    z = plsc.bitcast(val_b[pl.ds(gv * 16, 16)], jnp.float32)
                    z = (z - mean_z) * inv
                    addr = (v * 16 + lane) * 3
                    plsc.store_scatter(stage_v, [addr], x)
                    plsc.store_scatter(stage_v, [addr + 1], y)
                    plsc.store_scatter(stage_v, [addr + 2], z)
                    rx = x * r00 + y * r10 + z * r20
                    ry = x * r01 + y * r11 + z * r21
                    rz = x * r02 + y * r12 + z * r22
                    plsc.store_scatter(stage_r, [addr], rx)
                    plsc.store_scatter(stage_r, [addr + 1], ry)
                    plsc.store_scatter(stage_r, [addr + 2], rz)

                base = r * _RW + ch * _CW
                pltpu.sync_copy(stage_v, view_hbm.at[pl.ds(base, _CW)])
                pltpu.sync_copy(stage_r, viewr_hbm.at[pl.ds(base, _CW)])


def _sort_finalize(d, pts_t, crow):
    dkeys = lax.bitcast_convert_type(d.reshape(_B * 2 * _N), jnp.int32)
    mesh = plsc.VectorSubcoreMesh(core_axis_name="c", subcore_axis_name="s")
    view, viewr, means = pl.kernel(
        _sc_body,
        out_type=[
            jax.ShapeDtypeStruct((_B * 2 * _RW,), jnp.float32),
            jax.ShapeDtypeStruct((_B * 2 * _RW,), jnp.float32),
            jax.ShapeDtypeStruct((_B * 2 * 16,), jnp.float32),
        ],
        mesh=mesh,
        compiler_params=pltpu.CompilerParams(needs_layout_passes=False),
        scratch_types=[
            pltpu.VMEM((_N,), jnp.int32),
            pltpu.VMEM((_N,), jnp.int32),
            pltpu.VMEM((_N,), jnp.int32),
            pltpu.VMEM((_N,), jnp.int32),
            pltpu.VMEM((_N,), jnp.float32),
            pltpu.VMEM((_N,), jnp.float32),
            pltpu.VMEM((_N,), jnp.float32),
            pltpu.VMEM((4096,), jnp.int32),
            pltpu.VMEM((_CW,), jnp.float32),
            pltpu.VMEM((_CW,), jnp.float32),
            pltpu.VMEM((16,), jnp.float32),
            pltpu.VMEM((144,), jnp.float32),
        ],
    )(dkeys, pts_t.reshape(_B * 3 * _N), crow)
    return view, viewr, means


def kernel(pts):
    base = jax.random.key(42)
    k1, k2, k3, k4 = jax.random.split(base, 4)
    ci1 = jax.random.randint(k1, (_B,), 0, _N)
    ci2 = jax.random.randint(k2, (_B,), 0, _N)
    rot1 = _rotation_matrices(k3, _B)
    rot2 = _rotation_matrices(k4, _B)
    rots = jnp.stack([rot1, rot2], axis=1)
    crow = jnp.broadcast_to(
        rots.reshape(_B * 2, 9)[:, :, None], (_B * 2, 9, 16)).reshape(-1)

    c1 = jnp.take_along_axis(
        pts, jnp.broadcast_to(ci1[:, None, None], (_B, 1, 3)), axis=1)
    c2 = jnp.take_along_axis(
        pts, jnp.broadcast_to(ci2[:, None, None], (_B, 1, 3)), axis=1)
    centers = jnp.concatenate([c1, c2], axis=1)

    pts_t = pts.transpose(0, 2, 1)
    d = _distances(pts_t, centers)
    view, viewr, means = _sort_finalize(d, pts_t, crow)

    view = view.reshape(_B, 2, _PM, 3)
    viewr = viewr.reshape(_B, 2, _PM, 3)
    means = means.reshape(_B, 2, 16)[:, :, :3]
    relative_center = means[:, 1] - means[:, 0]
    view1 = view[:, 0, :_M, :]
    view2 = view[:, 1, :_M, :]
    view1_rot = viewr[:, 0, :_M, :]
    view2_rot = viewr[:, 1, :_M, :]
    return (relative_center, (view1_rot, view1), (view2_rot, view2))


# speedup vs baseline: 8.7067x; 5.0106x over previous
import functools

import jax
import jax.numpy as jnp
import numpy as np
from jax import lax
from jax.experimental import pallas as pl
from jax.experimental.pallas import tpu as pltpu
from jax.experimental.pallas import tpu_sc as plsc

_B = 64
_N = 16384
_M = 13107
_PM = 13120
_PMV = _PM // 16
_CVV = 82
_NCH = _PMV // _CVV
_CW = _CVV * 48
_RW = _PM * 3
_NV = _N // 16


def _rotation_matrices(key, b):
    ang = jax.random.uniform(key, (b, 3), minval=0.0, maxval=2.0 * np.pi)
    ca, sa = jnp.cos(ang), jnp.sin(ang)
    cx, cy, cz = ca[:, 0], ca[:, 1], ca[:, 2]
    sx, sy, sz = sa[:, 0], sa[:, 1], sa[:, 2]
    one = jnp.ones_like(cx)
    zero = jnp.zeros_like(cx)
    Rx = jnp.stack([jnp.stack([one, zero, zero], -1),
                    jnp.stack([zero, cx, -sx], -1),
                    jnp.stack([zero, sx, cx], -1)], -2)
    Ry = jnp.stack([jnp.stack([cy, zero, sy], -1),
                    jnp.stack([zero, one, zero], -1),
                    jnp.stack([-sy, zero, cy], -1)], -2)
    Rz = jnp.stack([jnp.stack([cz, -sz, zero], -1),
                    jnp.stack([sz, cz, zero], -1),
                    jnp.stack([zero, zero, one], -1)], -2)
    return jnp.einsum('bij,bjk,bkl->bil', Rz, Ry, Rx)


def _dist_body(centers_ref, pts_ref, d_ref):
    b = pl.program_id(0)
    for crop in range(2):
        cx = centers_ref[b, crop, 0]
        cy = centers_ref[b, crop, 1]
        cz = centers_ref[b, crop, 2]
        dx = pts_ref[0, 0:1, :] - cx
        dy = pts_ref[0, 1:2, :] - cy
        dz = pts_ref[0, 2:3, :] - cz
        d = jnp.sqrt(dx * dx + dy * dy + dz * dz)
        d_ref[0, crop:crop + 1, :] = d


def _distances(pts_t, centers):
    return pl.pallas_call(
        _dist_body,
        grid=(_B,),
        in_specs=[
            pl.BlockSpec(memory_space=pltpu.SMEM),
            pl.BlockSpec((1, 3, _N), lambda b: (b, 0, 0)),
        ],
        out_specs=pl.BlockSpec((1, 2, _N), lambda b: (b, 0, 0)),
        out_shape=jax.ShapeDtypeStruct((_B, 2, _N), jnp.float32),
    )(centers, pts_t)


def _sc_body(dkeys_hbm, pts_hbm, crow_hbm, view_hbm, viewr_hbm, means_hbm,
             key_a, key_b, val_a, val_b, ptsx, ptsy, ptsz, hist,
             cbuf, rotbuf):
    lane = lax.broadcasted_iota(jnp.int32, (16,), 0)
    ones = jnp.ones((16,), jnp.int32)
    fzero = jnp.zeros((16,), jnp.float32)
    wid = lax.axis_index("s") * 2 + lax.axis_index("c")

    def radix_pass(kin, vin, kout, vout, shift, last):
        @pl.loop(0, 256, unroll=4)
        def _(i):
            hist[pl.ds(i * 16, 16)] = jnp.zeros((16,), jnp.int32)

        @pl.loop(0, _NV, unroll=4)
        def _(v):
            k = kin[pl.ds(v * 16, 16)]
            dig = lax.shift_right_logical(k, shift) & 255
            plsc.addupdate_scatter(hist, [dig * 16 + lane], ones)

        @pl.loop(0, 256, init_carry=jnp.int32(0), unroll=2)
        def _(i, carry):
            h = hist[pl.ds(i * 16, 16)]
            c = plsc.cumsum(h)
            hist[pl.ds(i * 16, 16)] = c - h + carry
            return carry + jnp.sum(h)

        @pl.loop(0, _NV, unroll=4)
        def _(v):
            k = kin[pl.ds(v * 16, 16)]
            val = vin[pl.ds(v * 16, 16)]
            dig = lax.shift_right_logical(k, shift) & 255
            addr = dig * 16 + lane
            pos = plsc.load_gather(hist, [addr])
            plsc.addupdate_scatter(hist, [addr], ones)
            if last:
                plsc.store_scatter(vout, [pos], val)
            else:
                paddr = (pos & 1023) * 16 + lax.shift_right_logical(pos, 10)
                plsc.store_scatter(kout, [paddr], k)
                plsc.store_scatter(vout, [paddr], val)

    for j in range(2):
        b = wid * 2 + j
        for crop in range(2):
            r = b * 2 + crop
            pltpu.sync_copy(pts_hbm.at[pl.ds((b * 3 + 0) * _N, _N)], ptsx)
            pltpu.sync_copy(pts_hbm.at[pl.ds((b * 3 + 1) * _N, _N)], ptsy)
            pltpu.sync_copy(pts_hbm.at[pl.ds((b * 3 + 2) * _N, _N)], ptsz)
            pltpu.sync_copy(dkeys_hbm.at[pl.ds(r * _N, _N)], key_b)
            pltpu.sync_copy(crow_hbm.at[pl.ds(r * 144, 144)], rotbuf)

            @pl.loop(0, _NV, unroll=4)
            def _(v):
                src = lane * 1024 + ((v + lane) & 1023)
                vec = plsc.load_gather(key_b, [src])
                plsc.store_scatter(key_a, [((v + lane) & 1023) * 16 + lane],
                                   vec)

            @pl.loop(0, _NV, unroll=4)
            def _(v):
                val_a[pl.ds(v * 16, 16)] = lane * 1024 + v

            radix_pass(key_a, val_a, key_b, val_b, 0, False)
            radix_pass(key_b, val_b, key_a, val_a, 8, False)
            radix_pass(key_a, val_a, key_b, val_b, 16, False)
            radix_pass(key_b, val_b, None, val_a, 24, True)

            @pl.loop(0, _PMV, init_carry=(fzero, fzero, fzero), unroll=4)
            def s1(v, carry):
                sx, sy, sz = carry
                idx = val_a[pl.ds(v * 16, 16)]
                x = plsc.load_gather(ptsx, [idx])
                y = plsc.load_gather(ptsy, [idx])
                z = plsc.load_gather(ptsz, [idx])
                key_a[pl.ds(v * 16, 16)] = plsc.bitcast(x, jnp.int32)
                key_b[pl.ds(v * 16, 16)] = plsc.bitcast(y, jnp.int32)
                val_b[pl.ds(v * 16, 16)] = plsc.bitcast(z, jnp.int32)
                valid = (v * 16 + lane) < _M
                return (sx + jnp.where(valid, x, 0.0),
                        sy + jnp.where(valid, y, 0.0),
                        sz + jnp.where(valid, z, 0.0))

            sx, sy, sz = s1
            mean_x = jnp.sum(sx) * (1.0 / _M)
            mean_y = jnp.sum(sy) * (1.0 / _M)
            mean_z = jnp.sum(sz) * (1.0 / _M)

            cbuf[pl.ds(0, 16)] = jnp.where(
                lane == 0, mean_x, jnp.where(lane == 1, mean_y, mean_z))
            pltpu.sync_copy(cbuf, means_hbm.at[pl.ds(r * 16, 16)])

            @pl.loop(0, _PMV, init_carry=fzero, unroll=4)
            def s2(v, carry):
                x = plsc.bitcast(key_a[pl.ds(v * 16, 16)], jnp.float32)
                x = x - mean_x
                y = plsc.bitcast(key_b[pl.ds(v * 16, 16)], jnp.float32)
                y = y - mean_y
                z = plsc.bitcast(val_b[pl.ds(v * 16, 16)], jnp.float32)
                z = z - mean_z
                n2 = x * x + y * y + z * z
                valid = (v * 16 + lane) < _M
                return jnp.maximum(carry, jnp.where(valid, n2, 0.0))

            r2 = jnp.max(s2)
            r2v = fzero + r2
            yv = plsc.bitcast(
                jnp.int32(0x5F3759DF)
                - lax.shift_right_logical(plsc.bitcast(r2v, jnp.int32), 1),
                jnp.float32)
            yv = yv * (1.5 - 0.5 * r2v * yv * yv)
            yv = yv * (1.5 - 0.5 * r2v * yv * yv)
            yv = yv * (1.5 - 0.5 * r2v * yv * yv)
            inv = 1.0 / (r2v * yv + 1e-12)

            r00 = rotbuf[pl.ds(0, 16)]
            r01 = rotbuf[pl.ds(16, 16)]
            r02 = rotbuf[pl.ds(32, 16)]
            r10 = rotbuf[pl.ds(48, 16)]
            r11 = rotbuf[pl.ds(64, 16)]
            r12 = rotbuf[pl.ds(80, 16)]
            r20 = rotbuf[pl.ds(96, 16)]
            r21 = rotbuf[pl.ds(112, 16)]
            r22 = rotbuf[pl.ds(128, 16)]

            @pl.loop(0, _PMV, unroll=4)
            def _(v):
                sl = pl.ds(v * 16, 16)
                x = (plsc.bitcast(key_a[sl], jnp.float32) - mean_x) * inv
                y = (plsc.bitcast(key_b[sl], jnp.float32) - mean_y) * inv
                z = (plsc.bitcast(val_b[sl], jnp.float32) - mean_z) * inv
                ptsx[sl] = x
                ptsy[sl] = y
                ptsz[sl] = z

            base = r * 3 * _PM
            pltpu.sync_copy(ptsx.at[pl.ds(0, _PM)],
                            view_hbm.at[pl.ds(base, _PM)])
            pltpu.sync_copy(ptsy.at[pl.ds(0, _PM)],
                            view_hbm.at[pl.ds(base + _PM, _PM)])
            pltpu.sync_copy(ptsz.at[pl.ds(0, _PM)],
                            view_hbm.at[pl.ds(base + 2 * _PM, _PM)])

            @pl.loop(0, _PMV, unroll=4)
            def _(v):
                sl = pl.ds(v * 16, 16)
                x = (plsc.bitcast(key_a[sl], jnp.float32) - mean_x) * inv
                y = (plsc.bitcast(key_b[sl], jnp.float32) - mean_y) * inv
                z = (plsc.bitcast(val_b[sl], jnp.float32) - mean_z) * inv
                ptsx[sl] = x * r00 + y * r10 + z * r20
                ptsy[sl] = x * r01 + y * r11 + z * r21
                ptsz[sl] = x * r02 + y * r12 + z * r22

            pltpu.sync_copy(ptsx.at[pl.ds(0, _PM)],
                            viewr_hbm.at[pl.ds(base, _PM)])
            pltpu.sync_copy(ptsy.at[pl.ds(0, _PM)],
                            viewr_hbm.at[pl.ds(base + _PM, _PM)])
            pltpu.sync_copy(ptsz.at[pl.ds(0, _PM)],
                            viewr_hbm.at[pl.ds(base + 2 * _PM, _PM)])


def _sort_finalize(d, pts_t, crow):
    dkeys = lax.bitcast_convert_type(d.reshape(_B * 2 * _N), jnp.int32)
    mesh = plsc.VectorSubcoreMesh(core_axis_name="c", subcore_axis_name="s")
    view, viewr, means = pl.kernel(
        _sc_body,
        out_type=[
            jax.ShapeDtypeStruct((_B * 2 * 3 * _PM,), jnp.float32),
            jax.ShapeDtypeStruct((_B * 2 * 3 * _PM,), jnp.float32),
            jax.ShapeDtypeStruct((_B * 2 * 16,), jnp.float32),
        ],
        mesh=mesh,
        compiler_params=pltpu.CompilerParams(needs_layout_passes=False),
        scratch_types=[
            pltpu.VMEM((_N,), jnp.int32),
            pltpu.VMEM((_N,), jnp.int32),
            pltpu.VMEM((_N,), jnp.int32),
            pltpu.VMEM((_N,), jnp.int32),
            pltpu.VMEM((_N,), jnp.float32),
            pltpu.VMEM((_N,), jnp.float32),
            pltpu.VMEM((_N,), jnp.float32),
            pltpu.VMEM((4096,), jnp.int32),
            pltpu.VMEM((16,), jnp.float32),
            pltpu.VMEM((144,), jnp.float32),
        ],
    )(dkeys, pts_t.reshape(_B * 3 * _N), crow)
    return view, viewr, means


def kernel(pts):
    base = jax.random.key(42)
    k1, k2, k3, k4 = jax.random.split(base, 4)
    ci1 = jax.random.randint(k1, (_B,), 0, _N)
    ci2 = jax.random.randint(k2, (_B,), 0, _N)
    rot1 = _rotation_matrices(k3, _B)
    rot2 = _rotation_matrices(k4, _B)
    rots = jnp.stack([rot1, rot2], axis=1)
    crow = jnp.broadcast_to(
        rots.reshape(_B * 2, 9)[:, :, None], (_B * 2, 9, 16)).reshape(-1)

    c1 = jnp.take_along_axis(
        pts, jnp.broadcast_to(ci1[:, None, None], (_B, 1, 3)), axis=1)
    c2 = jnp.take_along_axis(
        pts, jnp.broadcast_to(ci2[:, None, None], (_B, 1, 3)), axis=1)
    centers = jnp.concatenate([c1, c2], axis=1)

    pts_t = pts.transpose(0, 2, 1)
    d = _distances(pts_t, centers)
    view, viewr, means = _sort_finalize(d, pts_t, crow)

    view = view.reshape(_B, 2, 3, _PM)
    viewr = viewr.reshape(_B, 2, 3, _PM)
    means = means.reshape(_B, 2, 16)[:, :, :3]
    relative_center = means[:, 1] - means[:, 0]
    view1 = view[:, 0, :, :_M].transpose(0, 2, 1)
    view2 = view[:, 1, :, :_M].transpose(0, 2, 1)
    view1_rot = viewr[:, 0, :, :_M].transpose(0, 2, 1)
    view2_rot = viewr[:, 1, :, :_M].transpose(0, 2, 1)
    return (relative_center, (view1_rot, view1), (view2_rot, view2))


# speedup vs baseline: 8.7208x; 1.0016x over previous
import functools

import jax
import jax.numpy as jnp
import numpy as np
from jax import lax
from jax.experimental import pallas as pl
from jax.experimental.pallas import tpu as pltpu
from jax.experimental.pallas import tpu_sc as plsc

_B = 64
_N = 16384
_M = 13107
_PM = 13120
_PMV = _PM // 16
_CVV = 82
_NCH = _PMV // _CVV
_CW = _CVV * 48
_RW = _PM * 3
_NV = _N // 16


def _rotation_matrices(key, b):
    ang = jax.random.uniform(key, (b, 3), minval=0.0, maxval=2.0 * np.pi)
    ca, sa = jnp.cos(ang), jnp.sin(ang)
    cx, cy, cz = ca[:, 0], ca[:, 1], ca[:, 2]
    sx, sy, sz = sa[:, 0], sa[:, 1], sa[:, 2]
    one = jnp.ones_like(cx)
    zero = jnp.zeros_like(cx)
    Rx = jnp.stack([jnp.stack([one, zero, zero], -1),
                    jnp.stack([zero, cx, -sx], -1),
                    jnp.stack([zero, sx, cx], -1)], -2)
    Ry = jnp.stack([jnp.stack([cy, zero, sy], -1),
                    jnp.stack([zero, one, zero], -1),
                    jnp.stack([-sy, zero, cy], -1)], -2)
    Rz = jnp.stack([jnp.stack([cz, -sz, zero], -1),
                    jnp.stack([sz, cz, zero], -1),
                    jnp.stack([zero, zero, one], -1)], -2)
    return jnp.einsum('bij,bjk,bkl->bil', Rz, Ry, Rx)


def _dist_body(centers_ref, pts_ref, d_ref):
    b = pl.program_id(0)
    for crop in range(2):
        cx = centers_ref[b, crop, 0]
        cy = centers_ref[b, crop, 1]
        cz = centers_ref[b, crop, 2]
        dx = pts_ref[0, 0:1, :] - cx
        dy = pts_ref[0, 1:2, :] - cy
        dz = pts_ref[0, 2:3, :] - cz
        d = jnp.sqrt(dx * dx + dy * dy + dz * dz)
        d_ref[0, crop:crop + 1, :] = d


def _distances(pts_t, centers):
    return pl.pallas_call(
        _dist_body,
        grid=(_B,),
        in_specs=[
            pl.BlockSpec(memory_space=pltpu.SMEM),
            pl.BlockSpec((1, 3, _N), lambda b: (b, 0, 0)),
        ],
        out_specs=pl.BlockSpec((1, 2, _N), lambda b: (b, 0, 0)),
        out_shape=jax.ShapeDtypeStruct((_B, 2, _N), jnp.float32),
    )(centers, pts_t)


def _sc_body(dkeys_hbm, pts_hbm, crow_hbm, view_hbm, viewr_hbm, means_hbm,
             key_a, key_b, val_a, val_b, ptsx, ptsy, ptsz, hist,
             cbuf, rotbuf):
    lane = lax.broadcasted_iota(jnp.int32, (16,), 0)
    ones = jnp.ones((16,), jnp.int32)
    fzero = jnp.zeros((16,), jnp.float32)
    wid = lax.axis_index("s") * 2 + lax.axis_index("c")

    def radix_pass(kin, vin, kout, vout, shift, last):
        @pl.loop(0, 256, unroll=4)
        def _(i):
            hist[pl.ds(i * 16, 16)] = jnp.zeros((16,), jnp.int32)

        @pl.loop(0, _NV, unroll=8)
        def _(v):
            k = kin[pl.ds(v * 16, 16)]
            dig = lax.shift_right_logical(k, shift) & 255
            plsc.addupdate_scatter(hist, [dig * 16 + lane], ones)

        @pl.loop(0, 256, init_carry=jnp.int32(0), unroll=2)
        def _(i, carry):
            h = hist[pl.ds(i * 16, 16)]
            c = plsc.cumsum(h)
            hist[pl.ds(i * 16, 16)] = c - h + carry
            return carry + jnp.sum(h)

        @pl.loop(0, _NV, unroll=8)
        def _(v):
            k = kin[pl.ds(v * 16, 16)]
            val = vin[pl.ds(v * 16, 16)]
            dig = lax.shift_right_logical(k, shift) & 255
            addr = dig * 16 + lane
            pos = plsc.load_gather(hist, [addr])
            plsc.addupdate_scatter(hist, [addr], ones)
            if last:
                plsc.store_scatter(vout, [pos], val)
            else:
                paddr = (pos & 1023) * 16 + lax.shift_right_logical(pos, 10)
                plsc.store_scatter(kout, [paddr], k)
                plsc.store_scatter(vout, [paddr], val)

    for j in range(2):
        b = wid * 2 + j
        for crop in range(2):
            r = b * 2 + crop
            pltpu.sync_copy(pts_hbm.at[pl.ds((b * 3 + 0) * _N, _N)], ptsx)
            pltpu.sync_copy(pts_hbm.at[pl.ds((b * 3 + 1) * _N, _N)], ptsy)
            pltpu.sync_copy(pts_hbm.at[pl.ds((b * 3 + 2) * _N, _N)], ptsz)
            pltpu.sync_copy(dkeys_hbm.at[pl.ds(r * _N, _N)], key_b)
            pltpu.sync_copy(crow_hbm.at[pl.ds(r * 144, 144)], rotbuf)

            @pl.loop(0, _NV, unroll=8)
            def _(v):
                src = lane * 1024 + ((v + lane) & 1023)
                vec = plsc.load_gather(key_b, [src])
                plsc.store_scatter(key_a, [((v + lane) & 1023) * 16 + lane],
                                   vec)

            @pl.loop(0, _NV, unroll=8)
            def _(v):
                val_a[pl.ds(v * 16, 16)] = lane * 1024 + v

            radix_pass(key_a, val_a, key_b, val_b, 0, False)
            radix_pass(key_b, val_b, key_a, val_a, 8, False)
            radix_pass(key_a, val_a, key_b, val_b, 16, False)
            radix_pass(key_b, val_b, None, val_a, 24, True)

            @pl.loop(0, _PMV, init_carry=(fzero, fzero, fzero), unroll=4)
            def s1(v, carry):
                sx, sy, sz = carry
                idx = val_a[pl.ds(v * 16, 16)]
                x = plsc.load_gather(ptsx, [idx])
                y = plsc.load_gather(ptsy, [idx])
                z = plsc.load_gather(ptsz, [idx])
                key_a[pl.ds(v * 16, 16)] = plsc.bitcast(x, jnp.int32)
                key_b[pl.ds(v * 16, 16)] = plsc.bitcast(y, jnp.int32)
                val_b[pl.ds(v * 16, 16)] = plsc.bitcast(z, jnp.int32)
                valid = (v * 16 + lane) < _M
                return (sx + jnp.where(valid, x, 0.0),
                        sy + jnp.where(valid, y, 0.0),
                        sz + jnp.where(valid, z, 0.0))

            sx, sy, sz = s1
            mean_x = jnp.sum(sx) * (1.0 / _M)
            mean_y = jnp.sum(sy) * (1.0 / _M)
            mean_z = jnp.sum(sz) * (1.0 / _M)

            cbuf[pl.ds(0, 16)] = jnp.where(
                lane == 0, mean_x, jnp.where(lane == 1, mean_y, mean_z))
            pltpu.sync_copy(cbuf, means_hbm.at[pl.ds(r * 16, 16)])

            @pl.loop(0, _PMV, init_carry=fzero, unroll=4)
            def s2(v, carry):
                x = plsc.bitcast(key_a[pl.ds(v * 16, 16)], jnp.float32)
                x = x - mean_x
                y = plsc.bitcast(key_b[pl.ds(v * 16, 16)], jnp.float32)
                y = y - mean_y
                z = plsc.bitcast(val_b[pl.ds(v * 16, 16)], jnp.float32)
                z = z - mean_z
                n2 = x * x + y * y + z * z
                valid = (v * 16 + lane) < _M
                return jnp.maximum(carry, jnp.where(valid, n2, 0.0))

            r2 = jnp.max(s2)
            r2v = fzero + r2
            yv = plsc.bitcast(
                jnp.int32(0x5F3759DF)
                - lax.shift_right_logical(plsc.bitcast(r2v, jnp.int32), 1),
                jnp.float32)
            yv = yv * (1.5 - 0.5 * r2v * yv * yv)
            yv = yv * (1.5 - 0.5 * r2v * yv * yv)
            yv = yv * (1.5 - 0.5 * r2v * yv * yv)
            inv = 1.0 / (r2v * yv + 1e-12)

            r00 = rotbuf[pl.ds(0, 16)]
            r01 = rotbuf[pl.ds(16, 16)]
            r02 = rotbuf[pl.ds(32, 16)]
            r10 = rotbuf[pl.ds(48, 16)]
            r11 = rotbuf[pl.ds(64, 16)]
            r12 = rotbuf[pl.ds(80, 16)]
            r20 = rotbuf[pl.ds(96, 16)]
            r21 = rotbuf[pl.ds(112, 16)]
            r22 = rotbuf[pl.ds(128, 16)]

            @pl.loop(0, _PMV, unroll=8)
            def _(v):
                sl = pl.ds(v * 16, 16)
                x = (plsc.bitcast(key_a[sl], jnp.float32) - mean_x) * inv
                y = (plsc.bitcast(key_b[sl], jnp.float32) - mean_y) * inv
                z = (plsc.bitcast(val_b[sl], jnp.float32) - mean_z) * inv
                ptsx[sl] = x
                ptsy[sl] = y
                ptsz[sl] = z

            base = r * 3 * _PM
            pltpu.sync_copy(ptsx.at[pl.ds(0, _PM)],
                            view_hbm.at[pl.ds(base, _PM)])
            pltpu.sync_copy(ptsy.at[pl.ds(0, _PM)],
                            view_hbm.at[pl.ds(base + _PM, _PM)])
            pltpu.sync_copy(ptsz.at[pl.ds(0, _PM)],
                            view_hbm.at[pl.ds(base + 2 * _PM, _PM)])

            @pl.loop(0, _PMV, unroll=8)
            def _(v):
                sl = pl.ds(v * 16, 16)
                x = (plsc.bitcast(key_a[sl], jnp.float32) - mean_x) * inv
                y = (plsc.bitcast(key_b[sl], jnp.float32) - mean_y) * inv
                z = (plsc.bitcast(val_b[sl], jnp.float32) - mean_z) * inv
                ptsx[sl] = x * r00 + y * r10 + z * r20
                ptsy[sl] = x * r01 + y * r11 + z * r21
                ptsz[sl] = x * r02 + y * r12 + z * r22

            pltpu.sync_copy(ptsx.at[pl.ds(0, _PM)],
                            viewr_hbm.at[pl.ds(base, _PM)])
            pltpu.sync_copy(ptsy.at[pl.ds(0, _PM)],
                            viewr_hbm.at[pl.ds(base + _PM, _PM)])
            pltpu.sync_copy(ptsz.at[pl.ds(0, _PM)],
                            viewr_hbm.at[pl.ds(base + 2 * _PM, _PM)])


def _sort_finalize(d, pts_t, crow):
    dkeys = lax.bitcast_convert_type(d.reshape(_B * 2 * _N), jnp.int32)
    mesh = plsc.VectorSubcoreMesh(core_axis_name="c", subcore_axis_name="s")
    view, viewr, means = pl.kernel(
        _sc_body,
        out_type=[
            jax.ShapeDtypeStruct((_B * 2 * 3 * _PM,), jnp.float32),
            jax.ShapeDtypeStruct((_B * 2 * 3 * _PM,), jnp.float32),
            jax.ShapeDtypeStruct((_B * 2 * 16,), jnp.float32),
        ],
        mesh=mesh,
        compiler_params=pltpu.CompilerParams(needs_layout_passes=False),
        scratch_types=[
            pltpu.VMEM((_N,), jnp.int32),
            pltpu.VMEM((_N,), jnp.int32),
            pltpu.VMEM((_N,), jnp.int32),
            pltpu.VMEM((_N,), jnp.int32),
            pltpu.VMEM((_N,), jnp.float32),
            pltpu.VMEM((_N,), jnp.float32),
            pltpu.VMEM((_N,), jnp.float32),
            pltpu.VMEM((4096,), jnp.int32),
            pltpu.VMEM((16,), jnp.float32),
            pltpu.VMEM((144,), jnp.float32),
        ],
    )(dkeys, pts_t.reshape(_B * 3 * _N), crow)
    return view, viewr, means


def kernel(pts):
    base = jax.random.key(42)
    k1, k2, k3, k4 = jax.random.split(base, 4)
    ci1 = jax.random.randint(k1, (_B,), 0, _N)
    ci2 = jax.random.randint(k2, (_B,), 0, _N)
    rot1 = _rotation_matrices(k3, _B)
    rot2 = _rotation_matrices(k4, _B)
    rots = jnp.stack([rot1, rot2], axis=1)
    crow = jnp.broadcast_to(
        rots.reshape(_B * 2, 9)[:, :, None], (_B * 2, 9, 16)).reshape(-1)

    c1 = jnp.take_along_axis(
        pts, jnp.broadcast_to(ci1[:, None, None], (_B, 1, 3)), axis=1)
    c2 = jnp.take_along_axis(
        pts, jnp.broadcast_to(ci2[:, None, None], (_B, 1, 3)), axis=1)
    centers = jnp.concatenate([c1, c2], axis=1)

    pts_t = pts.transpose(0, 2, 1)
    d = _distances(pts_t, centers)
    view, viewr, means = _sort_finalize(d, pts_t, crow)

    view = view.reshape(_B, 2, 3, _PM)
    viewr = viewr.reshape(_B, 2, 3, _PM)
    means = means.reshape(_B, 2, 16)[:, :, :3]
    relative_center = means[:, 1] - means[:, 0]
    view1 = view[:, 0, :, :_M].transpose(0, 2, 1)
    view2 = view[:, 1, :, :_M].transpose(0, 2, 1)
    view1_rot = viewr[:, 0, :, :_M].transpose(0, 2, 1)
    view2_rot = viewr[:, 1, :, :_M].transpose(0, 2, 1)
    return (relative_center, (view1_rot, view1), (view2_rot, view2))


# speedup vs baseline: 9.3390x; 1.0709x over previous
import functools

import jax
import jax.numpy as jnp
import numpy as np
from jax import lax
from jax.experimental import pallas as pl
from jax.experimental.pallas import tpu as pltpu
from jax.experimental.pallas import tpu_sc as plsc

_B = 64
_N = 16384
_M = 13107
_MP = 13184


def _rotation_matrices(key, b):
    ang = jax.random.uniform(key, (b, 3), minval=0.0, maxval=2.0 * np.pi)
    ca, sa = jnp.cos(ang), jnp.sin(ang)
    cx, cy, cz = ca[:, 0], ca[:, 1], ca[:, 2]
    sx, sy, sz = sa[:, 0], sa[:, 1], sa[:, 2]
    one = jnp.ones_like(cx)
    zero = jnp.zeros_like(cx)
    Rx = jnp.stack([jnp.stack([one, zero, zero], -1),
                    jnp.stack([zero, cx, -sx], -1),
                    jnp.stack([zero, sx, cx], -1)], -2)
    Ry = jnp.stack([jnp.stack([cy, zero, sy], -1),
                    jnp.stack([zero, one, zero], -1),
                    jnp.stack([-sy, zero, cy], -1)], -2)
    Rz = jnp.stack([jnp.stack([cz, -sz, zero], -1),
                    jnp.stack([sz, cz, zero], -1),
                    jnp.stack([zero, zero, one], -1)], -2)
    return jnp.einsum('bij,bjk,bkl->bil', Rz, Ry, Rx)


def _dist_body(centers_ref, pts_ref, d_ref):
    b = pl.program_id(0)
    for crop in range(2):
        cx = centers_ref[b, crop, 0]
        cy = centers_ref[b, crop, 1]
        cz = centers_ref[b, crop, 2]
        dx = pts_ref[0, 0:1, :] - cx
        dy = pts_ref[0, 1:2, :] - cy
        dz = pts_ref[0, 2:3, :] - cz
        d = jnp.sqrt(dx * dx + dy * dy + dz * dz)
        d_ref[0, crop:crop + 1, :] = d


def _distances(pts_t, centers):
    return pl.pallas_call(
        _dist_body,
        grid=(_B,),
        in_specs=[
            pl.BlockSpec(memory_space=pltpu.SMEM),
            pl.BlockSpec((1, 3, _N), lambda b: (b, 0, 0)),
        ],
        out_specs=pl.BlockSpec((1, 2, _N), lambda b: (b, 0, 0)),
        out_shape=jax.ShapeDtypeStruct((_B, 2, _N), jnp.float32),
    )(centers, pts_t)


def _finalize_body(rot_ref, sel_ref, view_ref, viewr_ref, rel_ref):
    b = pl.program_id(0)
    lane = lax.broadcasted_iota(jnp.int32, (1, _MP), 1)
    valid = lane < _M
    means = []
    for crop in range(2):
        rows = []
        msum = []
        for comp in range(3):
            row = sel_ref[0, crop, comp:comp + 1, :]
            s = jnp.sum(jnp.where(valid, row, 0.0))
            m = s * (1.0 / _M)
            msum.append(m)
            rows.append(row - m)
        means.append(msum)
        n2 = rows[0] * rows[0] + rows[1] * rows[1] + rows[2] * rows[2]
        r2 = jnp.max(jnp.where(valid, n2, 0.0))
        denom = jnp.sqrt(r2) + 1e-12
        vrows = [r / denom for r in rows]
        for comp in range(3):
            view_ref[0, crop, comp:comp + 1, :] = vrows[comp]
        for e in range(3):
            vr = (vrows[0] * rot_ref[b, crop, 0, e]
                  + vrows[1] * rot_ref[b, crop, 1, e]
                  + vrows[2] * rot_ref[b, crop, 2, e])
            viewr_ref[0, crop, e:e + 1, :] = vr
    col = lax.broadcasted_iota(jnp.int32, (1, 8, 128), 2)
    rel = jnp.where(
        col == 0, means[1][0] - means[0][0],
        jnp.where(col == 1, means[1][1] - means[0][1],
                  means[1][2] - means[0][2]))
    rel_ref[0, :, :] = rel[0]


def _finalize(sel, rots):
    return pl.pallas_call(
        _finalize_body,
        grid=(_B,),
        in_specs=[
            pl.BlockSpec(memory_space=pltpu.SMEM),
            pl.BlockSpec((1, 2, 3, _MP), lambda b: (b, 0, 0, 0)),
        ],
        out_specs=[
            pl.BlockSpec((1, 2, 3, _MP), lambda b: (b, 0, 0, 0)),
            pl.BlockSpec((1, 2, 3, _MP), lambda b: (b, 0, 0, 0)),
            pl.BlockSpec((1, 8, 128), lambda b: (b, 0, 0)),
        ],
        out_shape=[
            jax.ShapeDtypeStruct((_B, 2, 3, _MP), jnp.float32),
            jax.ShapeDtypeStruct((_B, 2, 3, _MP), jnp.float32),
            jax.ShapeDtypeStruct((_B, 8, 128), jnp.float32),
        ],
    )(rots, sel)


_NV = _N // 16
_CHUNK = 1648
_NCHUNK = _MP // _CHUNK
_CV = _CHUNK // 16


def _sc_sort_body(dkeys_hbm, pts_hbm, sel_hbm,
                  key_a, key_b, val_a, val_b,
                  ptsx, ptsy, ptsz, hist, selx, sely, selz):
    lane = lax.broadcasted_iota(jnp.int32, (16,), 0)
    ones = jnp.ones((16,), jnp.int32)
    wid = lax.axis_index("s") * 2 + lax.axis_index("c")

    def radix_pass(kin, vin, kout, vout, shift, last):
        @pl.loop(0, 256, unroll=4)
        def _(i):
            hist[pl.ds(i * 16, 16)] = jnp.zeros((16,), jnp.int32)

        @pl.loop(0, _NV, unroll=4)
        def _(v):
            k = kin[pl.ds(v * 16, 16)]
            dig = lax.shift_right_logical(k, shift) & 255
            plsc.addupdate_scatter(hist, [dig * 16 + lane], ones)

        @pl.loop(0, 256, init_carry=jnp.int32(0), unroll=2)
        def _(i, carry):
            h = hist[pl.ds(i * 16, 16)]
            c = plsc.cumsum(h)
            hist[pl.ds(i * 16, 16)] = c - h + carry
            return carry + jnp.sum(h)

        @pl.loop(0, _NV, unroll=4)
        def _(v):
            k = kin[pl.ds(v * 16, 16)]
            val = vin[pl.ds(v * 16, 16)]
            dig = lax.shift_right_logical(k, shift) & 255
            addr = dig * 16 + lane
            pos = plsc.load_gather(hist, [addr])
            plsc.addupdate_scatter(hist, [addr], ones)
            if last:
                plsc.store_scatter(vout, [pos], val)
            else:
                paddr = (pos & 1023) * 16 + lax.shift_right_logical(pos, 10)
                plsc.store_scatter(kout, [paddr], k)
                plsc.store_scatter(vout, [paddr], val)

    for j in range(2):
        b = wid * 2 + j
        pltpu.sync_copy(pts_hbm.at[pl.ds((b * 3 + 0) * _N, _N)], ptsx)
        pltpu.sync_copy(pts_hbm.at[pl.ds((b * 3 + 1) * _N, _N)], ptsy)
        pltpu.sync_copy(pts_hbm.at[pl.ds((b * 3 + 2) * _N, _N)], ptsz)
        for crop in range(2):
            r = b * 2 + crop
            pltpu.sync_copy(dkeys_hbm.at[pl.ds(r * _N, _N)], key_b)

            @pl.loop(0, _NV, unroll=4)
            def _(v):
                src = lane * 1024 + ((v + lane) & 1023)
                vec = plsc.load_gather(key_b, [src])
                plsc.store_scatter(key_a, [((v + lane) & 1023) * 16 + lane],
                                   vec)

            @pl.loop(0, _NV, unroll=4)
            def _(v):
                val_a[pl.ds(v * 16, 16)] = lane * 1024 + v

            radix_pass(key_a, val_a, key_b, val_b, 0, False)
            radix_pass(key_b, val_b, key_a, val_a, 8, False)
            radix_pass(key_a, val_a, key_b, val_b, 16, False)
            radix_pass(key_b, val_b, key_a, val_a, 24, True)

            for ch in range(_NCHUNK):
                @pl.loop(0, _CV, unroll=4)
                def _(v):
                    idx = val_a[pl.ds((ch * _CV + v) * 16, 16)]
                    selx[pl.ds(v * 16, 16)] = plsc.load_gather(ptsx, [idx])
                    sely[pl.ds(v * 16, 16)] = plsc.load_gather(ptsy, [idx])
                    selz[pl.ds(v * 16, 16)] = plsc.load_gather(ptsz, [idx])
                off = ch * _CHUNK
                pltpu.sync_copy(
                    selx, sel_hbm.at[pl.ds((r * 3 + 0) * _MP + off, _CHUNK)])
                pltpu.sync_copy(
                    sely, sel_hbm.at[pl.ds((r * 3 + 1) * _MP + off, _CHUNK)])
                pltpu.sync_copy(
                    selz, sel_hbm.at[pl.ds((r * 3 + 2) * _MP + off, _CHUNK)])


def _sort_gather(d, pts_t):
    dkeys = lax.bitcast_convert_type(d.reshape(_B * 2 * _N), jnp.int32)
    mesh = plsc.VectorSubcoreMesh(core_axis_name="c", subcore_axis_name="s")
    sel = pl.kernel(
        _sc_sort_body,
        out_type=jax.ShapeDtypeStruct((_B * 2 * 3 * _MP,), jnp.float32),
        mesh=mesh,
        compiler_params=pltpu.CompilerParams(needs_layout_passes=False),
        scratch_types=[
            pltpu.VMEM((_N,), jnp.int32),
            pltpu.VMEM((_N,), jnp.int32),
            pltpu.VMEM((_N,), jnp.int32),
            pltpu.VMEM((_N,), jnp.int32),
            pltpu.VMEM((_N,), jnp.float32),
            pltpu.VMEM((_N,), jnp.float32),
            pltpu.VMEM((_N,), jnp.float32),
            pltpu.VMEM((4096,), jnp.int32),
            pltpu.VMEM((_CHUNK,), jnp.float32),
            pltpu.VMEM((_CHUNK,), jnp.float32),
            pltpu.VMEM((_CHUNK,), jnp.float32),
        ],
    )(dkeys, pts_t.reshape(_B * 3 * _N))
    return sel.reshape(_B, 2, 3, _MP)


def kernel(pts):
    base = jax.random.key(42)
    k1, k2, k3, k4 = jax.random.split(base, 4)
    ci1 = jax.random.randint(k1, (_B,), 0, _N)
    ci2 = jax.random.randint(k2, (_B,), 0, _N)
    rot1 = _rotation_matrices(k3, _B)
    rot2 = _rotation_matrices(k4, _B)
    rots = jnp.stack([rot1, rot2], axis=1)

    c1 = jnp.take_along_axis(
        pts, jnp.broadcast_to(ci1[:, None, None], (_B, 1, 3)), axis=1)
    c2 = jnp.take_along_axis(
        pts, jnp.broadcast_to(ci2[:, None, None], (_B, 1, 3)), axis=1)
    centers = jnp.concatenate([c1, c2], axis=1)

    pts_t = pts.transpose(0, 2, 1)
    d = _distances(pts_t, centers)
    sel = _sort_gather(d, pts_t)
    view, viewr, rel = _finalize(sel, rots)

    relative_center = rel[:, 0, :3]
    view1 = view[:, 0, :, :_M].transpose(0, 2, 1)
    view2 = view[:, 1, :, :_M].transpose(0, 2, 1)
    view1_rot = viewr[:, 0, :, :_M].transpose(0, 2, 1)
    view2_rot = viewr[:, 1, :, :_M].transpose(0, 2, 1)
    return (relative_center, (view1_rot, view1), (view2_rot, view2))


# speedup vs baseline: 10.0459x; 1.0757x over previous
import functools

import jax
import jax.numpy as jnp
import numpy as np
from jax import lax
from jax.experimental import pallas as pl
from jax.experimental.pallas import tpu as pltpu
from jax.experimental.pallas import tpu_sc as plsc

_B = 64
_N = 16384
_M = 13107
_MP = 13184


def _rotation_matrices(key, b):
    ang = jax.random.uniform(key, (b, 3), minval=0.0, maxval=2.0 * np.pi)
    ca, sa = jnp.cos(ang), jnp.sin(ang)
    cx, cy, cz = ca[:, 0], ca[:, 1], ca[:, 2]
    sx, sy, sz = sa[:, 0], sa[:, 1], sa[:, 2]
    one = jnp.ones_like(cx)
    zero = jnp.zeros_like(cx)
    Rx = jnp.stack([jnp.stack([one, zero, zero], -1),
                    jnp.stack([zero, cx, -sx], -1),
                    jnp.stack([zero, sx, cx], -1)], -2)
    Ry = jnp.stack([jnp.stack([cy, zero, sy], -1),
                    jnp.stack([zero, one, zero], -1),
                    jnp.stack([-sy, zero, cy], -1)], -2)
    Rz = jnp.stack([jnp.stack([cz, -sz, zero], -1),
                    jnp.stack([sz, cz, zero], -1),
                    jnp.stack([zero, zero, one], -1)], -2)
    return jnp.einsum('bij,bjk,bkl->bil', Rz, Ry, Rx)


def _dist_body(centers_ref, pts_ref, d_ref):
    b = pl.program_id(0)
    for crop in range(2):
        cx = centers_ref[b, crop, 0]
        cy = centers_ref[b, crop, 1]
        cz = centers_ref[b, crop, 2]
        dx = pts_ref[0, 0:1, :] - cx
        dy = pts_ref[0, 1:2, :] - cy
        dz = pts_ref[0, 2:3, :] - cz
        d = jnp.sqrt(dx * dx + dy * dy + dz * dz)
        d_ref[0, crop:crop + 1, :] = d


def _distances(pts_t, centers):
    return pl.pallas_call(
        _dist_body,
        grid=(_B,),
        in_specs=[
            pl.BlockSpec(memory_space=pltpu.SMEM),
            pl.BlockSpec((1, 3, _N), lambda b: (b, 0, 0)),
        ],
        out_specs=pl.BlockSpec((1, 2, _N), lambda b: (b, 0, 0)),
        out_shape=jax.ShapeDtypeStruct((_B, 2, _N), jnp.float32),
    )(centers, pts_t)


def _finalize_body(rot_ref, sel_ref, view_ref, viewr_ref, rel_ref):
    b = pl.program_id(0)
    lane = lax.broadcasted_iota(jnp.int32, (1, _MP), 1)
    valid = lane < _M
    means = []
    for crop in range(2):
        rows = []
        msum = []
        for comp in range(3):
            row = sel_ref[0, crop, comp:comp + 1, :]
            s = jnp.sum(jnp.where(valid, row, 0.0))
            m = s * (1.0 / _M)
            msum.append(m)
            rows.append(row - m)
        means.append(msum)
        n2 = rows[0] * rows[0] + rows[1] * rows[1] + rows[2] * rows[2]
        r2 = jnp.max(jnp.where(valid, n2, 0.0))
        denom = jnp.sqrt(r2) + 1e-12
        vrows = [r / denom for r in rows]
        for comp in range(3):
            view_ref[0, crop, comp:comp + 1, :] = vrows[comp]
        for e in range(3):
            vr = (vrows[0] * rot_ref[b, crop, 0, e]
                  + vrows[1] * rot_ref[b, crop, 1, e]
                  + vrows[2] * rot_ref[b, crop, 2, e])
            viewr_ref[0, crop, e:e + 1, :] = vr
    col = lax.broadcasted_iota(jnp.int32, (1, 8, 128), 2)
    rel = jnp.where(
        col == 0, means[1][0] - means[0][0],
        jnp.where(col == 1, means[1][1] - means[0][1],
                  means[1][2] - means[0][2]))
    rel_ref[0, :, :] = rel[0]


def _finalize(sel, rots):
    return pl.pallas_call(
        _finalize_body,
        grid=(_B,),
        in_specs=[
            pl.BlockSpec(memory_space=pltpu.SMEM),
            pl.BlockSpec((1, 2, 3, _MP), lambda b: (b, 0, 0, 0)),
        ],
        out_specs=[
            pl.BlockSpec((1, 2, 3, _MP), lambda b: (b, 0, 0, 0)),
            pl.BlockSpec((1, 2, 3, _MP), lambda b: (b, 0, 0, 0)),
            pl.BlockSpec((1, 8, 128), lambda b: (b, 0, 0)),
        ],
        out_shape=[
            jax.ShapeDtypeStruct((_B, 2, 3, _MP), jnp.float32),
            jax.ShapeDtypeStruct((_B, 2, 3, _MP), jnp.float32),
            jax.ShapeDtypeStruct((_B, 8, 128), jnp.float32),
        ],
    )(rots, sel)


_NV = _N // 16
_CHUNK = 1648
_NCHUNK = _MP // _CHUNK
_CV = _CHUNK // 16


def _sc_sort_body(dkeys_hbm, pts_hbm, sel_hbm,
                  key_a, key_b, val_a, val_b,
                  ptsx, ptsy, ptsz, hist, selx, sely, selz):
    lane = lax.broadcasted_iota(jnp.int32, (16,), 0)
    ones = jnp.ones((16,), jnp.int32)
    wid = lax.axis_index("s") * 2 + lax.axis_index("c")

    def radix_pass(kin, vin, kout, vout, shift, last):
        @pl.loop(0, 256, unroll=4)
        def _(i):
            hist[pl.ds(i * 16, 16)] = jnp.zeros((16,), jnp.int32)

        @pl.loop(0, _NV, unroll=4)
        def _(v):
            k = kin[pl.ds(v * 16, 16)]
            dig = lax.shift_right_logical(k, shift) & 255
            plsc.addupdate_scatter(hist, [dig * 16 + lane], ones)

        @pl.loop(0, 256, init_carry=jnp.int32(0), unroll=2)
        def _(i, carry):
            h = hist[pl.ds(i * 16, 16)]
            c = plsc.cumsum(h)
            hist[pl.ds(i * 16, 16)] = c - h + carry
            return carry + jnp.sum(h)

        @pl.loop(0, _NV, unroll=4)
        def _(v):
            k = kin[pl.ds(v * 16, 16)]
            val = vin[pl.ds(v * 16, 16)]
            dig = lax.shift_right_logical(k, shift) & 255
            addr = dig * 16 + lane
            pos = plsc.load_gather(hist, [addr])
            plsc.addupdate_scatter(hist, [addr], ones)
            if last:
                plsc.store_scatter(vout, [pos], val)
            else:
                paddr = (pos & 1023) * 16 + lax.shift_right_logical(pos, 10)
                plsc.store_scatter(kout, [paddr], k)
                plsc.store_scatter(vout, [paddr], val)

    for j in range(2):
        b = wid * 2 + j
        pltpu.sync_copy(pts_hbm.at[pl.ds((b * 3 + 0) * _N, _N)], ptsx)
        pltpu.sync_copy(pts_hbm.at[pl.ds((b * 3 + 1) * _N, _N)], ptsy)
        pltpu.sync_copy(pts_hbm.at[pl.ds((b * 3 + 2) * _N, _N)], ptsz)
        for crop in range(2):
            r = b * 2 + crop
            pltpu.sync_copy(dkeys_hbm.at[pl.ds(r * _N, _N)], key_b)

            @plsc.parallel_loop(0, _NV, unroll=4)
            def _(v):
                src = lane * 1024 + ((v + lane) & 1023)
                vec = plsc.load_gather(key_b, [src])
                plsc.store_scatter(key_a, [((v + lane) & 1023) * 16 + lane],
                                   vec)

            @plsc.parallel_loop(0, _NV, unroll=4)
            def _(v):
                val_a[pl.ds(v * 16, 16)] = lane * 1024 + v

            radix_pass(key_a, val_a, key_b, val_b, 0, False)
            radix_pass(key_b, val_b, key_a, val_a, 8, False)
            radix_pass(key_a, val_a, key_b, val_b, 16, False)
            radix_pass(key_b, val_b, key_a, val_a, 24, True)

            for ch in range(_NCHUNK):
                @plsc.parallel_loop(0, _CV, unroll=4)
                def _(v):
                    idx = val_a[pl.ds((ch * _CV + v) * 16, 16)]
                    selx[pl.ds(v * 16, 16)] = plsc.load_gather(ptsx, [idx])
                    sely[pl.ds(v * 16, 16)] = plsc.load_gather(ptsy, [idx])
                    selz[pl.ds(v * 16, 16)] = plsc.load_gather(ptsz, [idx])
                off = ch * _CHUNK
                pltpu.sync_copy(
                    selx, sel_hbm.at[pl.ds((r * 3 + 0) * _MP + off, _CHUNK)])
                pltpu.sync_copy(
                    sely, sel_hbm.at[pl.ds((r * 3 + 1) * _MP + off, _CHUNK)])
                pltpu.sync_copy(
                    selz, sel_hbm.at[pl.ds((r * 3 + 2) * _MP + off, _CHUNK)])


def _sort_gather(d, pts_t):
    dkeys = lax.bitcast_convert_type(d.reshape(_B * 2 * _N), jnp.int32)
    mesh = plsc.VectorSubcoreMesh(core_axis_name="c", subcore_axis_name="s")
    sel = pl.kernel(
        _sc_sort_body,
        out_type=jax.ShapeDtypeStruct((_B * 2 * 3 * _MP,), jnp.float32),
        mesh=mesh,
        compiler_params=pltpu.CompilerParams(needs_layout_passes=False),
        scratch_types=[
            pltpu.VMEM((_N,), jnp.int32),
            pltpu.VMEM((_N,), jnp.int32),
            pltpu.VMEM((_N,), jnp.int32),
            pltpu.VMEM((_N,), jnp.int32),
            pltpu.VMEM((_N,), jnp.float32),
            pltpu.VMEM((_N,), jnp.float32),
            pltpu.VMEM((_N,), jnp.float32),
            pltpu.VMEM((4096,), jnp.int32),
            pltpu.VMEM((_CHUNK,), jnp.float32),
            pltpu.VMEM((_CHUNK,), jnp.float32),
            pltpu.VMEM((_CHUNK,), jnp.float32),
        ],
    )(dkeys, pts_t.reshape(_B * 3 * _N))
    return sel.reshape(_B, 2, 3, _MP)


def kernel(pts):
    base = jax.random.key(42)
    k1, k2, k3, k4 = jax.random.split(base, 4)
    ci1 = jax.random.randint(k1, (_B,), 0, _N)
    ci2 = jax.random.randint(k2, (_B,), 0, _N)
    rot1 = _rotation_matrices(k3, _B)
    rot2 = _rotation_matrices(k4, _B)
    rots = jnp.stack([rot1, rot2], axis=1)

    c1 = jnp.take_along_axis(
        pts, jnp.broadcast_to(ci1[:, None, None], (_B, 1, 3)), axis=1)
    c2 = jnp.take_along_axis(
        pts, jnp.broadcast_to(ci2[:, None, None], (_B, 1, 3)), axis=1)
    centers = jnp.concatenate([c1, c2], axis=1)

    pts_t = pts.transpose(0, 2, 1)
    d = _distances(pts_t, centers)
    sel = _sort_gather(d, pts_t)
    view, viewr, rel = _finalize(sel, rots)

    relative_center = rel[:, 0, :3]
    view1 = view[:, 0, :, :_M].transpose(0, 2, 1)
    view2 = view[:, 1, :, :_M].transpose(0, 2, 1)
    view1_rot = viewr[:, 0, :, :_M].transpose(0, 2, 1)
    view2_rot = viewr[:, 1, :, :_M].transpose(0, 2, 1)
    return (relative_center, (view1_rot, view1), (view2_rot, view2))


# speedup vs baseline: 11.7617x; 1.1708x over previous
import functools

import jax
import jax.numpy as jnp
import numpy as np
from jax import lax
from jax.experimental import pallas as pl
from jax.experimental.pallas import tpu as pltpu
from jax.experimental.pallas import tpu_sc as plsc

_B = 64
_N = 16384
_M = 13107
_MP = 13184


def _rotation_matrices(key, b):
    ang = jax.random.uniform(key, (b, 3), minval=0.0, maxval=2.0 * np.pi)
    ca, sa = jnp.cos(ang), jnp.sin(ang)
    cx, cy, cz = ca[:, 0], ca[:, 1], ca[:, 2]
    sx, sy, sz = sa[:, 0], sa[:, 1], sa[:, 2]
    one = jnp.ones_like(cx)
    zero = jnp.zeros_like(cx)
    Rx = jnp.stack([jnp.stack([one, zero, zero], -1),
                    jnp.stack([zero, cx, -sx], -1),
                    jnp.stack([zero, sx, cx], -1)], -2)
    Ry = jnp.stack([jnp.stack([cy, zero, sy], -1),
                    jnp.stack([zero, one, zero], -1),
                    jnp.stack([-sy, zero, cy], -1)], -2)
    Rz = jnp.stack([jnp.stack([cz, -sz, zero], -1),
                    jnp.stack([sz, cz, zero], -1),
                    jnp.stack([zero, zero, one], -1)], -2)
    return jnp.einsum('bij,bjk,bkl->bil', Rz, Ry, Rx)


def _dist_body(centers_ref, pts_ref, d_ref):
    b = pl.program_id(0)
    for crop in range(2):
        cx = centers_ref[b, crop, 0]
        cy = centers_ref[b, crop, 1]
        cz = centers_ref[b, crop, 2]
        dx = pts_ref[0, 0:1, :] - cx
        dy = pts_ref[0, 1:2, :] - cy
        dz = pts_ref[0, 2:3, :] - cz
        d = jnp.sqrt(dx * dx + dy * dy + dz * dz)
        d_ref[0, crop:crop + 1, :] = d


def _distances(pts_t, centers):
    return pl.pallas_call(
        _dist_body,
        grid=(_B,),
        in_specs=[
            pl.BlockSpec(memory_space=pltpu.SMEM),
            pl.BlockSpec((1, 3, _N), lambda b: (b, 0, 0)),
        ],
        out_specs=pl.BlockSpec((1, 2, _N), lambda b: (b, 0, 0)),
        out_shape=jax.ShapeDtypeStruct((_B, 2, _N), jnp.float32),
    )(centers, pts_t)


def _finalize_body(rot_ref, sel_ref, view_ref, viewr_ref, rel_ref):
    b = pl.program_id(0)
    lane = lax.broadcasted_iota(jnp.int32, (1, _MP), 1)
    valid = lane < _M
    means = []
    for crop in range(2):
        rows = []
        msum = []
        for comp in range(3):
            row = sel_ref[0, crop, comp:comp + 1, :]
            s = jnp.sum(jnp.where(valid, row, 0.0))
            m = s * (1.0 / _M)
            msum.append(m)
            rows.append(row - m)
        means.append(msum)
        n2 = rows[0] * rows[0] + rows[1] * rows[1] + rows[2] * rows[2]
        r2 = jnp.max(jnp.where(valid, n2, 0.0))
        denom = jnp.sqrt(r2) + 1e-12
        vrows = [r / denom for r in rows]
        for comp in range(3):
            view_ref[0, crop, comp:comp + 1, :] = vrows[comp]
        for e in range(3):
            vr = (vrows[0] * rot_ref[b, crop, 0, e]
                  + vrows[1] * rot_ref[b, crop, 1, e]
                  + vrows[2] * rot_ref[b, crop, 2, e])
            viewr_ref[0, crop, e:e + 1, :] = vr
    col = lax.broadcasted_iota(jnp.int32, (1, 8, 128), 2)
    rel = jnp.where(
        col == 0, means[1][0] - means[0][0],
        jnp.where(col == 1, means[1][1] - means[0][1],
                  means[1][2] - means[0][2]))
    rel_ref[0, :, :] = rel[0]


def _finalize(sel, rots):
    return pl.pallas_call(
        _finalize_body,
        grid=(_B,),
        in_specs=[
            pl.BlockSpec(memory_space=pltpu.SMEM),
            pl.BlockSpec((1, 2, 3, _MP), lambda b: (b, 0, 0, 0)),
        ],
        out_specs=[
            pl.BlockSpec((1, 2, 3, _MP), lambda b: (b, 0, 0, 0)),
            pl.BlockSpec((1, 2, 3, _MP), lambda b: (b, 0, 0, 0)),
            pl.BlockSpec((1, 8, 128), lambda b: (b, 0, 0)),
        ],
        out_shape=[
            jax.ShapeDtypeStruct((_B, 2, 3, _MP), jnp.float32),
            jax.ShapeDtypeStruct((_B, 2, 3, _MP), jnp.float32),
            jax.ShapeDtypeStruct((_B, 8, 128), jnp.float32),
        ],
    )(rots, sel)


_NV = _N // 16
_CHUNK = 1648
_NCHUNK = _MP // _CHUNK
_CV = _CHUNK // 16


def _sc_sort_body(dkeys_hbm, pts_hbm, sel_hbm,
                  key_a, key_b, val_a, val_b,
                  ptsx, ptsy, ptsz, hist, selx, sely, selz):
    lane = lax.broadcasted_iota(jnp.int32, (16,), 0)
    ones = jnp.ones((16,), jnp.int32)
    wid = lax.axis_index("s") * 2 + lax.axis_index("c")

    def radix_pass(kin, vin, kout, vout, shift, last):
        @plsc.parallel_loop(0, 256, unroll=4)
        def _(i):
            hist[pl.ds(i * 16, 16)] = jnp.zeros((16,), jnp.int32)

        @plsc.parallel_loop(0, _NV, unroll=4)
        def _(v):
            k = kin[pl.ds(v * 16, 16)]
            dig = lax.shift_right_logical(k, shift) & 255
            plsc.addupdate_scatter(hist, [dig * 16 + lane], ones)

        @pl.loop(0, 256, init_carry=jnp.int32(0), unroll=2)
        def _(i, carry):
            h = hist[pl.ds(i * 16, 16)]
            c = plsc.cumsum(h)
            hist[pl.ds(i * 16, 16)] = c - h + carry
            return carry + jnp.sum(h)

        @pl.loop(0, _NV, unroll=4)
        def _(v):
            k = kin[pl.ds(v * 16, 16)]
            val = vin[pl.ds(v * 16, 16)]
            dig = lax.shift_right_logical(k, shift) & 255
            addr = dig * 16 + lane
            pos = plsc.load_gather(hist, [addr])
            plsc.addupdate_scatter(hist, [addr], ones)
            if last:
                plsc.store_scatter(vout, [pos], val)
            else:
                paddr = (pos & 1023) * 16 + lax.shift_right_logical(pos, 10)
                plsc.store_scatter(kout, [paddr], k)
                plsc.store_scatter(vout, [paddr], val)

    for j in range(2):
        b = wid * 2 + j
        pltpu.sync_copy(pts_hbm.at[pl.ds((b * 3 + 0) * _N, _N)], ptsx)
        pltpu.sync_copy(pts_hbm.at[pl.ds((b * 3 + 1) * _N, _N)], ptsy)
        pltpu.sync_copy(pts_hbm.at[pl.ds((b * 3 + 2) * _N, _N)], ptsz)
        for crop in range(2):
            r = b * 2 + crop
            pltpu.sync_copy(dkeys_hbm.at[pl.ds(r * _N, _N)], key_b)

            @plsc.parallel_loop(0, _NV, unroll=4)
            def _(v):
                src = lane * 1024 + ((v + lane) & 1023)
                vec = plsc.load_gather(key_b, [src])
                plsc.store_scatter(key_a, [((v + lane) & 1023) * 16 + lane],
                                   vec)

            @plsc.parallel_loop(0, _NV, unroll=4)
            def _(v):
                val_a[pl.ds(v * 16, 16)] = lane * 1024 + v

            radix_pass(key_a, val_a, key_b, val_b, 0, False)
            radix_pass(key_b, val_b, key_a, val_a, 8, False)
            radix_pass(key_a, val_a, key_b, val_b, 16, False)
            radix_pass(key_b, val_b, key_a, val_a, 24, True)

            for ch in range(_NCHUNK):
                @plsc.parallel_loop(0, _CV, unroll=4)
                def _(v):
                    idx = val_a[pl.ds((ch * _CV + v) * 16, 16)]
                    selx[pl.ds(v * 16, 16)] = plsc.load_gather(ptsx, [idx])
                    sely[pl.ds(v * 16, 16)] = plsc.load_gather(ptsy, [idx])
                    selz[pl.ds(v * 16, 16)] = plsc.load_gather(ptsz, [idx])
                off = ch * _CHUNK
                pltpu.sync_copy(
                    selx, sel_hbm.at[pl.ds((r * 3 + 0) * _MP + off, _CHUNK)])
                pltpu.sync_copy(
                    sely, sel_hbm.at[pl.ds((r * 3 + 1) * _MP + off, _CHUNK)])
                pltpu.sync_copy(
                    selz, sel_hbm.at[pl.ds((r * 3 + 2) * _MP + off, _CHUNK)])


def _sort_gather(d, pts_t):
    dkeys = lax.bitcast_convert_type(d.reshape(_B * 2 * _N), jnp.int32)
    mesh = plsc.VectorSubcoreMesh(core_axis_name="c", subcore_axis_name="s")
    sel = pl.kernel(
        _sc_sort_body,
        out_type=jax.ShapeDtypeStruct((_B * 2 * 3 * _MP,), jnp.float32),
        mesh=mesh,
        compiler_params=pltpu.CompilerParams(needs_layout_passes=False),
        scratch_types=[
            pltpu.VMEM((_N,), jnp.int32),
            pltpu.VMEM((_N,), jnp.int32),
            pltpu.VMEM((_N,), jnp.int32),
            pltpu.VMEM((_N,), jnp.int32),
            pltpu.VMEM((_N,), jnp.float32),
            pltpu.VMEM((_N,), jnp.float32),
            pltpu.VMEM((_N,), jnp.float32),
            pltpu.VMEM((4096,), jnp.int32),
            pltpu.VMEM((_CHUNK,), jnp.float32),
            pltpu.VMEM((_CHUNK,), jnp.float32),
            pltpu.VMEM((_CHUNK,), jnp.float32),
        ],
    )(dkeys, pts_t.reshape(_B * 3 * _N))
    return sel.reshape(_B, 2, 3, _MP)


def kernel(pts):
    base = jax.random.key(42)
    k1, k2, k3, k4 = jax.random.split(base, 4)
    ci1 = jax.random.randint(k1, (_B,), 0, _N)
    ci2 = jax.random.randint(k2, (_B,), 0, _N)
    rot1 = _rotation_matrices(k3, _B)
    rot2 = _rotation_matrices(k4, _B)
    rots = jnp.stack([rot1, rot2], axis=1)

    c1 = jnp.take_along_axis(
        pts, jnp.broadcast_to(ci1[:, None, None], (_B, 1, 3)), axis=1)
    c2 = jnp.take_along_axis(
        pts, jnp.broadcast_to(ci2[:, None, None], (_B, 1, 3)), axis=1)
    centers = jnp.concatenate([c1, c2], axis=1)

    pts_t = pts.transpose(0, 2, 1)
    d = _distances(pts_t, centers)
    sel = _sort_gather(d, pts_t)
    view, viewr, rel = _finalize(sel, rots)

    relative_center = rel[:, 0, :3]
    view1 = view[:, 0, :, :_M].transpose(0, 2, 1)
    view2 = view[:, 1, :, :_M].transpose(0, 2, 1)
    view1_rot = viewr[:, 0, :, :_M].transpose(0, 2, 1)
    view2_rot = viewr[:, 1, :, :_M].transpose(0, 2, 1)
    return (relative_center, (view1_rot, view1), (view2_rot, view2))


# speedup vs baseline: 12.0005x; 1.0203x over previous
import functools

import jax
import jax.numpy as jnp
import numpy as np
from jax import lax
from jax.experimental import pallas as pl
from jax.experimental.pallas import tpu as pltpu
from jax.experimental.pallas import tpu_sc as plsc

_B = 64
_N = 16384
_M = 13107
_MP = 13184


def _rotation_matrices(key, b):
    ang = jax.random.uniform(key, (b, 3), minval=0.0, maxval=2.0 * np.pi)
    ca, sa = jnp.cos(ang), jnp.sin(ang)
    cx, cy, cz = ca[:, 0], ca[:, 1], ca[:, 2]
    sx, sy, sz = sa[:, 0], sa[:, 1], sa[:, 2]
    one = jnp.ones_like(cx)
    zero = jnp.zeros_like(cx)
    Rx = jnp.stack([jnp.stack([one, zero, zero], -1),
                    jnp.stack([zero, cx, -sx], -1),
                    jnp.stack([zero, sx, cx], -1)], -2)
    Ry = jnp.stack([jnp.stack([cy, zero, sy], -1),
                    jnp.stack([zero, one, zero], -1),
                    jnp.stack([-sy, zero, cy], -1)], -2)
    Rz = jnp.stack([jnp.stack([cz, -sz, zero], -1),
                    jnp.stack([sz, cz, zero], -1),
                    jnp.stack([zero, zero, one], -1)], -2)
    return jnp.einsum('bij,bjk,bkl->bil', Rz, Ry, Rx)


def _dist_body(centers_ref, pts_ref, d_ref):
    b = pl.program_id(0)
    for crop in range(2):
        cx = centers_ref[b, crop, 0]
        cy = centers_ref[b, crop, 1]
        cz = centers_ref[b, crop, 2]
        dx = pts_ref[0, 0:1, :] - cx
        dy = pts_ref[0, 1:2, :] - cy
        dz = pts_ref[0, 2:3, :] - cz
        d = jnp.sqrt(dx * dx + dy * dy + dz * dz)
        d_ref[0, crop:crop + 1, :] = d


def _distances(pts_t, centers):
    return pl.pallas_call(
        _dist_body,
        grid=(_B,),
        in_specs=[
            pl.BlockSpec(memory_space=pltpu.SMEM),
            pl.BlockSpec((1, 3, _N), lambda b: (b, 0, 0)),
        ],
        out_specs=pl.BlockSpec((1, 2, _N), lambda b: (b, 0, 0)),
        out_shape=jax.ShapeDtypeStruct((_B, 2, _N), jnp.float32),
    )(centers, pts_t)


def _finalize_body(rot_ref, sel_ref, view_ref, viewr_ref, rel_ref):
    b = pl.program_id(0)
    lane = lax.broadcasted_iota(jnp.int32, (1, _MP), 1)
    valid = lane < _M
    means = []
    for crop in range(2):
        rows = []
        msum = []
        for comp in range(3):
            row = sel_ref[0, crop, comp:comp + 1, :]
            s = jnp.sum(jnp.where(valid, row, 0.0))
            m = s * (1.0 / _M)
            msum.append(m)
            rows.append(row - m)
        means.append(msum)
        n2 = rows[0] * rows[0] + rows[1] * rows[1] + rows[2] * rows[2]
        r2 = jnp.max(jnp.where(valid, n2, 0.0))
        denom = jnp.sqrt(r2) + 1e-12
        vrows = [r / denom for r in rows]
        for comp in range(3):
            view_ref[0, crop, comp:comp + 1, :] = vrows[comp]
        for e in range(3):
            vr = (vrows[0] * rot_ref[b, crop, 0, e]
                  + vrows[1] * rot_ref[b, crop, 1, e]
                  + vrows[2] * rot_ref[b, crop, 2, e])
            viewr_ref[0, crop, e:e + 1, :] = vr
    col = lax.broadcasted_iota(jnp.int32, (1, 8, 128), 2)
    rel = jnp.where(
        col == 0, means[1][0] - means[0][0],
        jnp.where(col == 1, means[1][1] - means[0][1],
                  means[1][2] - means[0][2]))
    rel_ref[0, :, :] = rel[0]


def _finalize(sel, rots):
    return pl.pallas_call(
        _finalize_body,
        grid=(_B,),
        in_specs=[
            pl.BlockSpec(memory_space=pltpu.SMEM),
            pl.BlockSpec((1, 2, 3, _MP), lambda b: (b, 0, 0, 0)),
        ],
        out_specs=[
            pl.BlockSpec((1, 2, 3, _MP), lambda b: (b, 0, 0, 0)),
            pl.BlockSpec((1, 2, 3, _MP), lambda b: (b, 0, 0, 0)),
            pl.BlockSpec((1, 8, 128), lambda b: (b, 0, 0)),
        ],
        out_shape=[
            jax.ShapeDtypeStruct((_B, 2, 3, _MP), jnp.float32),
            jax.ShapeDtypeStruct((_B, 2, 3, _MP), jnp.float32),
            jax.ShapeDtypeStruct((_B, 8, 128), jnp.float32),
        ],
    )(rots, sel)


_NV = _N // 16
_CHUNK = 1648
_NCHUNK = _MP // _CHUNK
_CV = _CHUNK // 16


def _sc_sort_body(dkeys_hbm, pts_hbm, sel_hbm,
                  key_a, key_b, val_a, val_b,
                  ptsx, ptsy, ptsz, hist, selx, sely, selz):
    lane = lax.broadcasted_iota(jnp.int32, (16,), 0)
    ones = jnp.ones((16,), jnp.int32)
    wid = lax.axis_index("s") * 2 + lax.axis_index("c")

    def radix_pass(kin, vin, kout, vout, shift, last):
        @plsc.parallel_loop(0, 256, unroll=4)
        def _(i):
            hist[pl.ds(i * 16, 16)] = jnp.zeros((16,), jnp.int32)

        @plsc.parallel_loop(0, _NV, unroll=4)
        def _(v):
            k = kin[pl.ds(v * 16, 16)]
            dig = lax.shift_right_logical(k, shift) & 255
            plsc.addupdate_scatter(hist, [dig * 16 + lane], ones)

        @plsc.parallel_loop(0, 256, unroll=2, carry=jnp.int32(0))
        def _(i, carry):
            h = hist[pl.ds(i * 16, 16)]
            c = plsc.cumsum(h)
            hist[pl.ds(i * 16, 16)] = c - h + carry
            return carry + jnp.sum(h)

        @pl.loop(0, _NV, unroll=4)
        def _(v):
            k = kin[pl.ds(v * 16, 16)]
            val = vin[pl.ds(v * 16, 16)]
            dig = lax.shift_right_logical(k, shift) & 255
            addr = dig * 16 + lane
            pos = plsc.load_gather(hist, [addr])
            plsc.addupdate_scatter(hist, [addr], ones)
            if last:
                plsc.store_scatter(vout, [pos], val)
            else:
                paddr = (pos & 1023) * 16 + lax.shift_right_logical(pos, 10)
                plsc.store_scatter(kout, [paddr], k)
                plsc.store_scatter(vout, [paddr], val)

    for j in range(2):
        b = wid * 2 + j
        pltpu.sync_copy(pts_hbm.at[pl.ds((b * 3 + 0) * _N, _N)], ptsx)
        pltpu.sync_copy(pts_hbm.at[pl.ds((b * 3 + 1) * _N, _N)], ptsy)
        pltpu.sync_copy(pts_hbm.at[pl.ds((b * 3 + 2) * _N, _N)], ptsz)
        for crop in range(2):
            r = b * 2 + crop
            pltpu.sync_copy(dkeys_hbm.at[pl.ds(r * _N, _N)], key_b)

            @plsc.parallel_loop(0, _NV, unroll=4)
            def _(v):
                src = lane * 1024 + ((v + lane) & 1023)
                vec = plsc.load_gather(key_b, [src])
                plsc.store_scatter(key_a, [((v + lane) & 1023) * 16 + lane],
                                   vec)

            @plsc.parallel_loop(0, _NV, unroll=4)
            def _(v):
                val_a[pl.ds(v * 16, 16)] = lane * 1024 + v

            radix_pass(key_a, val_a, key_b, val_b, 0, False)
            radix_pass(key_b, val_b, key_a, val_a, 8, False)
            radix_pass(key_a, val_a, key_b, val_b, 16, False)
            radix_pass(key_b, val_b, key_a, val_a, 24, True)

            for ch in range(_NCHUNK):
                @plsc.parallel_loop(0, _CV, unroll=4)
                def _(v):
                    idx = val_a[pl.ds((ch * _CV + v) * 16, 16)]
                    selx[pl.ds(v * 16, 16)] = plsc.load_gather(ptsx, [idx])
                    sely[pl.ds(v * 16, 16)] = plsc.load_gather(ptsy, [idx])
                    selz[pl.ds(v * 16, 16)] = plsc.load_gather(ptsz, [idx])
                off = ch * _CHUNK
                pltpu.sync_copy(
                    selx, sel_hbm.at[pl.ds((r * 3 + 0) * _MP + off, _CHUNK)])
                pltpu.sync_copy(
                    sely, sel_hbm.at[pl.ds((r * 3 + 1) * _MP + off, _CHUNK)])
                pltpu.sync_copy(
                    selz, sel_hbm.at[pl.ds((r * 3 + 2) * _MP + off, _CHUNK)])


def _sort_gather(d, pts_t):
    dkeys = lax.bitcast_convert_type(d.reshape(_B * 2 * _N), jnp.int32)
    mesh = plsc.VectorSubcoreMesh(core_axis_name="c", subcore_axis_name="s")
    sel = pl.kernel(
        _sc_sort_body,
        out_type=jax.ShapeDtypeStruct((_B * 2 * 3 * _MP,), jnp.float32),
        mesh=mesh,
        compiler_params=pltpu.CompilerParams(needs_layout_passes=False),
        scratch_types=[
            pltpu.VMEM((_N,), jnp.int32),
            pltpu.VMEM((_N,), jnp.int32),
            pltpu.VMEM((_N,), jnp.int32),
            pltpu.VMEM((_N,), jnp.int32),
            pltpu.VMEM((_N,), jnp.float32),
            pltpu.VMEM((_N,), jnp.float32),
            pltpu.VMEM((_N,), jnp.float32),
            pltpu.VMEM((4096,), jnp.int32),
            pltpu.VMEM((_CHUNK,), jnp.float32),
            pltpu.VMEM((_CHUNK,), jnp.float32),
            pltpu.VMEM((_CHUNK,), jnp.float32),
        ],
    )(dkeys, pts_t.reshape(_B * 3 * _N))
    return sel.reshape(_B, 2, 3, _MP)


def kernel(pts):
    base = jax.random.key(42)
    k1, k2, k3, k4 = jax.random.split(base, 4)
    ci1 = jax.random.randint(k1, (_B,), 0, _N)
    ci2 = jax.random.randint(k2, (_B,), 0, _N)
    rot1 = _rotation_matrices(k3, _B)
    rot2 = _rotation_matrices(k4, _B)
    rots = jnp.stack([rot1, rot2], axis=1)

    c1 = jnp.take_along_axis(
        pts, jnp.broadcast_to(ci1[:, None, None], (_B, 1, 3)), axis=1)
    c2 = jnp.take_along_axis(
        pts, jnp.broadcast_to(ci2[:, None, None], (_B, 1, 3)), axis=1)
    centers = jnp.concatenate([c1, c2], axis=1)

    pts_t = pts.transpose(0, 2, 1)
    d = _distances(pts_t, centers)
    sel = _sort_gather(d, pts_t)
    view, viewr, rel = _finalize(sel, rots)

    relative_center = rel[:, 0, :3]
    view1 = view[:, 0, :, :_M].transpose(0, 2, 1)
    view2 = view[:, 1, :, :_M].transpose(0, 2, 1)
    view1_rot = viewr[:, 0, :, :_M].transpose(0, 2, 1)
    view2_rot = viewr[:, 1, :, :_M].transpose(0, 2, 1)
    return (relative_center, (view1_rot, view1), (view2_rot, view2))


# speedup vs baseline: 12.9954x; 1.0829x over previous
import functools

import jax
import jax.numpy as jnp
import numpy as np
from jax import lax
from jax.experimental import pallas as pl
from jax.experimental.pallas import tpu as pltpu
from jax.experimental.pallas import tpu_sc as plsc

_B = 64
_N = 16384
_M = 13107
_MP = 13184


def _rotation_matrices(key, b):
    ang = jax.random.uniform(key, (b, 3), minval=0.0, maxval=2.0 * np.pi)
    ca, sa = jnp.cos(ang), jnp.sin(ang)
    cx, cy, cz = ca[:, 0], ca[:, 1], ca[:, 2]
    sx, sy, sz = sa[:, 0], sa[:, 1], sa[:, 2]
    one = jnp.ones_like(cx)
    zero = jnp.zeros_like(cx)
    Rx = jnp.stack([jnp.stack([one, zero, zero], -1),
                    jnp.stack([zero, cx, -sx], -1),
                    jnp.stack([zero, sx, cx], -1)], -2)
    Ry = jnp.stack([jnp.stack([cy, zero, sy], -1),
                    jnp.stack([zero, one, zero], -1),
                    jnp.stack([-sy, zero, cy], -1)], -2)
    Rz = jnp.stack([jnp.stack([cz, -sz, zero], -1),
                    jnp.stack([sz, cz, zero], -1),
                    jnp.stack([zero, zero, one], -1)], -2)
    return jnp.einsum('bij,bjk,bkl->bil', Rz, Ry, Rx)


def _dist_body(centers_ref, pts_ref, d_ref):
    b = pl.program_id(0)
    for crop in range(2):
        cx = centers_ref[b, crop, 0]
        cy = centers_ref[b, crop, 1]
        cz = centers_ref[b, crop, 2]
        dx = pts_ref[0, 0:1, :] - cx
        dy = pts_ref[0, 1:2, :] - cy
        dz = pts_ref[0, 2:3, :] - cz
        d = jnp.sqrt(dx * dx + dy * dy + dz * dz)
        d_ref[0, crop:crop + 1, :] = d


def _distances(pts_t, centers):
    return pl.pallas_call(
        _dist_body,
        grid=(_B,),
        in_specs=[
            pl.BlockSpec(memory_space=pltpu.SMEM),
            pl.BlockSpec((1, 3, _N), lambda b: (b, 0, 0)),
        ],
        out_specs=pl.BlockSpec((1, 2, _N), lambda b: (b, 0, 0)),
        out_shape=jax.ShapeDtypeStruct((_B, 2, _N), jnp.float32),
    )(centers, pts_t)


def _finalize_body(rot_ref, sel_ref, view_ref, viewr_ref, rel_ref):
    b = pl.program_id(0)
    lane = lax.broadcasted_iota(jnp.int32, (1, _MP), 1)
    valid = lane < _M
    means = []
    for crop in range(2):
        rows = []
        msum = []
        for comp in range(3):
            row = sel_ref[0, crop, comp:comp + 1, :]
            s = jnp.sum(jnp.where(valid, row, 0.0))
            m = s * (1.0 / _M)
            msum.append(m)
            rows.append(row - m)
        means.append(msum)
        n2 = rows[0] * rows[0] + rows[1] * rows[1] + rows[2] * rows[2]
        r2 = jnp.max(jnp.where(valid, n2, 0.0))
        denom = jnp.sqrt(r2) + 1e-12
        vrows = [r / denom for r in rows]
        for comp in range(3):
            view_ref[0, crop, comp:comp + 1, :] = vrows[comp]
        for e in range(3):
            vr = (vrows[0] * rot_ref[b, crop, 0, e]
                  + vrows[1] * rot_ref[b, crop, 1, e]
                  + vrows[2] * rot_ref[b, crop, 2, e])
            viewr_ref[0, crop, e:e + 1, :] = vr
    col = lax.broadcasted_iota(jnp.int32, (1, 8, 128), 2)
    rel = jnp.where(
        col == 0, means[1][0] - means[0][0],
        jnp.where(col == 1, means[1][1] - means[0][1],
                  means[1][2] - means[0][2]))
    rel_ref[0, :, :] = rel[0]


def _finalize(sel, rots):
    return pl.pallas_call(
        _finalize_body,
        grid=(_B,),
        in_specs=[
            pl.BlockSpec(memory_space=pltpu.SMEM),
            pl.BlockSpec((1, 2, 3, _MP), lambda b: (b, 0, 0, 0)),
        ],
        out_specs=[
            pl.BlockSpec((1, 2, 3, _MP), lambda b: (b, 0, 0, 0)),
            pl.BlockSpec((1, 2, 3, _MP), lambda b: (b, 0, 0, 0)),
            pl.BlockSpec((1, 8, 128), lambda b: (b, 0, 0)),
        ],
        out_shape=[
            jax.ShapeDtypeStruct((_B, 2, 3, _MP), jnp.float32),
            jax.ShapeDtypeStruct((_B, 2, 3, _MP), jnp.float32),
            jax.ShapeDtypeStruct((_B, 8, 128), jnp.float32),
        ],
    )(rots, sel)


_NV = _N // 16
_CHUNK = 1648
_NCHUNK = _MP // _CHUNK
_CV = _CHUNK // 16


def _sc_sort_body(dkeys_hbm, pts_hbm, sel_hbm,
                  key_a, key_b, val_a, val_b,
                  ptsx, ptsy, ptsz, hist, hist2, selx, sely, selz):
    lane = lax.broadcasted_iota(jnp.int32, (16,), 0)
    ones = jnp.ones((16,), jnp.int32)
    wid = lax.axis_index("s") * 2 + lax.axis_index("c")

    _HV = _NV // 2

    def radix_pass(kin, vin, kout, vout, shift, last):
        @plsc.parallel_loop(0, 256, unroll=4)
        def _(i):
            hist[pl.ds(i * 16, 16)] = jnp.zeros((16,), jnp.int32)
            hist2[pl.ds(i * 16, 16)] = jnp.zeros((16,), jnp.int32)

        @plsc.parallel_loop(0, _HV, unroll=4)
        def _(v):
            k1 = kin[pl.ds(v * 16, 16)]
            d1 = lax.shift_right_logical(k1, shift) & 255
            plsc.addupdate_scatter(hist, [d1 * 16 + lane], ones)
            k2 = kin[pl.ds((v + _HV) * 16, 16)]
            d2 = lax.shift_right_logical(k2, shift) & 255
            plsc.addupdate_scatter(hist2, [d2 * 16 + lane], ones)

        @plsc.parallel_loop(0, 256, unroll=2, carry=jnp.int32(0))
        def _(i, carry):
            h1 = hist[pl.ds(i * 16, 16)]
            h2 = hist2[pl.ds(i * 16, 16)]
            s = h1 + h2
            c = plsc.cumsum(s)
            excl = c - s + carry
            hist[pl.ds(i * 16, 16)] = excl
            hist2[pl.ds(i * 16, 16)] = excl + h1
            return carry + jnp.sum(s)

        @pl.loop(0, _HV, unroll=2)
        def _(v):
            k1 = kin[pl.ds(v * 16, 16)]
            val1 = vin[pl.ds(v * 16, 16)]
            a1 = (lax.shift_right_logical(k1, shift) & 255) * 16 + lane
            pos1 = plsc.load_gather(hist, [a1])
            plsc.addupdate_scatter(hist, [a1], ones)
            k2 = kin[pl.ds((v + _HV) * 16, 16)]
            val2 = vin[pl.ds((v + _HV) * 16, 16)]
            a2 = (lax.shift_right_logical(k2, shift) & 255) * 16 + lane
            pos2 = plsc.load_gather(hist2, [a2])
            plsc.addupdate_scatter(hist2, [a2], ones)
            if last:
                plsc.store_scatter(vout, [pos1], val1)
                plsc.store_scatter(vout, [pos2], val2)
            else:
                p1 = (pos1 & 1023) * 16 + lax.shift_right_logical(pos1, 10)
                plsc.store_scatter(kout, [p1], k1)
                plsc.store_scatter(vout, [p1], val1)
                p2 = (pos2 & 1023) * 16 + lax.shift_right_logical(pos2, 10)
                plsc.store_scatter(kout, [p2], k2)
                plsc.store_scatter(vout, [p2], val2)

    for j in range(2):
        b = wid * 2 + j
        pltpu.sync_copy(pts_hbm.at[pl.ds((b * 3 + 0) * _N, _N)], ptsx)
        pltpu.sync_copy(pts_hbm.at[pl.ds((b * 3 + 1) * _N, _N)], ptsy)
        pltpu.sync_copy(pts_hbm.at[pl.ds((b * 3 + 2) * _N, _N)], ptsz)
        for crop in range(2):
            r = b * 2 + crop
            pltpu.sync_copy(dkeys_hbm.at[pl.ds(r * _N, _N)], key_b)

            @plsc.parallel_loop(0, _NV, unroll=4)
            def _(v):
                src = lane * 1024 + ((v + lane) & 1023)
                vec = plsc.load_gather(key_b, [src])
                plsc.store_scatter(key_a, [((v + lane) & 1023) * 16 + lane],
                                   vec)

            @plsc.parallel_loop(0, _NV, unroll=4)
            def _(v):
                val_a[pl.ds(v * 16, 16)] = lane * 1024 + v

            radix_pass(key_a, val_a, key_b, val_b, 0, False)
            radix_pass(key_b, val_b, key_a, val_a, 8, False)
            radix_pass(key_a, val_a, key_b, val_b, 16, False)
            radix_pass(key_b, val_b, key_a, val_a, 24, True)

            for ch in range(_NCHUNK):
                @plsc.parallel_loop(0, _CV, unroll=4)
                def _(v):
                    idx = val_a[pl.ds((ch * _CV + v) * 16, 16)]
                    selx[pl.ds(v * 16, 16)] = plsc.load_gather(ptsx, [idx])
                    sely[pl.ds(v * 16, 16)] = plsc.load_gather(ptsy, [idx])
                    selz[pl.ds(v * 16, 16)] = plsc.load_gather(ptsz, [idx])
                off = ch * _CHUNK
                pltpu.sync_copy(
                    selx, sel_hbm.at[pl.ds((r * 3 + 0) * _MP + off, _CHUNK)])
                pltpu.sync_copy(
                    sely, sel_hbm.at[pl.ds((r * 3 + 1) * _MP + off, _CHUNK)])
                pltpu.sync_copy(
                    selz, sel_hbm.at[pl.ds((r * 3 + 2) * _MP + off, _CHUNK)])


def _sort_gather(d, pts_t):
    dkeys = lax.bitcast_convert_type(d.reshape(_B * 2 * _N), jnp.int32)
    mesh = plsc.VectorSubcoreMesh(core_axis_name="c", subcore_axis_name="s")
    sel = pl.kernel(
        _sc_sort_body,
        out_type=jax.ShapeDtypeStruct((_B * 2 * 3 * _MP,), jnp.float32),
        mesh=mesh,
        compiler_params=pltpu.CompilerParams(needs_layout_passes=False),
        scratch_types=[
            pltpu.VMEM((_N,), jnp.int32),
            pltpu.VMEM((_N,), jnp.int32),
            pltpu.VMEM((_N,), jnp.int32),
            pltpu.VMEM((_N,), jnp.int32),
            pltpu.VMEM((_N,), jnp.float32),
            pltpu.VMEM((_N,), jnp.float32),
            pltpu.VMEM((_N,), jnp.float32),
            pltpu.VMEM((4096,), jnp.int32),
            pltpu.VMEM((4096,), jnp.int32),
            pltpu.VMEM((_CHUNK,), jnp.float32),
            pltpu.VMEM((_CHUNK,), jnp.float32),
            pltpu.VMEM((_CHUNK,), jnp.float32),
        ],
    )(dkeys, pts_t.reshape(_B * 3 * _N))
    return sel.reshape(_B, 2, 3, _MP)


def kernel(pts):
    base = jax.random.key(42)
    k1, k2, k3, k4 = jax.random.split(base, 4)
    ci1 = jax.random.randint(k1, (_B,), 0, _N)
    ci2 = jax.random.randint(k2, (_B,), 0, _N)
    rot1 = _rotation_matrices(k3, _B)
    rot2 = _rotation_matrices(k4, _B)
    rots = jnp.stack([rot1, rot2], axis=1)

    c1 = jnp.take_along_axis(
        pts, jnp.broadcast_to(ci1[:, None, None], (_B, 1, 3)), axis=1)
    c2 = jnp.take_along_axis(
        pts, jnp.broadcast_to(ci2[:, None, None], (_B, 1, 3)), axis=1)
    centers = jnp.concatenate([c1, c2], axis=1)

    pts_t = pts.transpose(0, 2, 1)
    d = _distances(pts_t, centers)
    sel = _sort_gather(d, pts_t)
    view, viewr, rel = _finalize(sel, rots)

    relative_center = rel[:, 0, :3]
    view1 = view[:, 0, :, :_M].transpose(0, 2, 1)
    view2 = view[:, 1, :, :_M].transpose(0, 2, 1)
    view1_rot = viewr[:, 0, :, :_M].transpose(0, 2, 1)
    view2_rot = viewr[:, 1, :, :_M].transpose(0, 2, 1)
    return (relative_center, (view1_rot, view1), (view2_rot, view2))
